# baseline (speedup 1.0000x reference)
import functools
from functools import partial

import jax
import jax.numpy as jnp
import numpy as np
from jax import lax
from jax.experimental import pallas as pl
from jax.experimental.pallas import tpu as pltpu
from jax.experimental.pallas import tpu_sc as plsc

N = 10000
D = 128
G = 64
GF = 16
H = 128
C = 10
K = 4
EPS = 1e-5

_INTERPRET = False

_NP = 10240
_RT = 1000


def _norm_body(x_ref, brow_ref, bcol_ref, ms_ref, w_ref, b_ref,
               h_ref, starts_ref):
    x = x_ref[...]
    brow = brow_ref[...]
    bcol = bcol_ref[...]
    g_row = jax.lax.broadcasted_iota(jnp.int32, (1, 128), 1)
    g_col = jax.lax.broadcasted_iota(jnp.int32, (128, 1), 0)
    oh = (brow == g_row).astype(jnp.float32)
    ohT = (g_col == bcol).astype(jnp.float32)
    cnt = jnp.sum(ohT, axis=1, keepdims=True)
    cnt = jnp.maximum(cnt, 1.0)
    sums = jax.lax.dot_general(ohT, x, (((1,), (0,)), ((), ())),
                               preferred_element_type=jnp.float32)
    mean = sums / cnt
    meanb = jax.lax.dot_general(oh, mean, (((1,), (0,)), ((), ())),
                                preferred_element_type=jnp.float32)
    xc = x - ms_ref[...] * meanb
    var = jax.lax.dot_general(ohT, xc * xc, (((1,), (0,)), ((), ())),
                              preferred_element_type=jnp.float32) / cnt
    std = jnp.sqrt(var + EPS)
    stdb = jax.lax.dot_general(oh, std, (((1,), (0,)), ((), ())),
                               preferred_element_type=jnp.float32)
    h_ref[...] = w_ref[...] * xc / stdb + b_ref[...]
    tri = (g_col < g_row).astype(jnp.float32)
    cnt_row = jnp.sum(oh, axis=0, keepdims=True)
    starts_f = jax.lax.dot_general(cnt_row, tri, (((1,), (0,)), ((), ())),
                                   preferred_element_type=jnp.float32)
    starts_ref[...] = starts_f.astype(jnp.int32)


def _graph_norm_starts(x, batch, p):
    brow = batch.reshape(N, 1)
    bcol = batch.reshape(1, N)
    ms = p['gn_ms'].reshape(1, D)
    w = p['gn_w'].reshape(1, D)
    b = p['gn_b'].reshape(1, D)
    h, starts = pl.pallas_call(
        _norm_body,
        out_shape=(jax.ShapeDtypeStruct((N, D), jnp.float32),
                   jax.ShapeDtypeStruct((1, 128), jnp.int32)),
        interpret=_INTERPRET,
    )(x, brow, bcol, ms, w, b)
    return h, starts[0, :G + 1]


def _proj_body(x_ref, w_ref, b_ref, q_ref, kv_ref, s_ref):
    y = jax.lax.dot_general(x_ref[...], w_ref[...], (((1,), (0,)), ((), ())),
                            preferred_element_type=jnp.float32) + b_ref[...]
    q_ref[...] = y[:, 0:128]
    kv_ref[...] = y[:, 128:384]
    s_ref[...] = y[:, 384:512]


def _projections(h, p):
    w4 = jnp.concatenate([p['tq_w'], p['tk_w'], p['tv_w'], p['ts_w']], axis=1)
    b4 = jnp.concatenate([p['tq_b'], p['tk_b'], p['tv_b'], p['ts_b']]
                         ).reshape(1, 512)
    outs = pl.pallas_call(
        _proj_body,
        grid=(N // _RT,),
        in_specs=[
            pl.BlockSpec((_RT, D), lambda i: (i, 0)),
            pl.BlockSpec((D, 512), lambda i: (0, 0)),
            pl.BlockSpec((1, 512), lambda i: (0, 0)),
        ],
        out_specs=(pl.BlockSpec((_RT, H), lambda i: (i, 0)),
                   pl.BlockSpec((_RT, 256), lambda i: (i, 0)),
                   pl.BlockSpec((_RT, H), lambda i: (i, 0))),
        out_shape=(jax.ShapeDtypeStruct((N, H), jnp.float32),
                   jax.ShapeDtypeStruct((N, 256), jnp.float32),
                   jax.ShapeDtypeStruct((N, H), jnp.float32)),
        interpret=_INTERPRET,
    )(h, w4, b4)
    return outs


def _gate_body(o_ref, s_ref, wa_ref, wb_ref, c_ref):
    o = o_ref[...]
    s = s_ref[...]
    z = jnp.sum(o * wa_ref[...] + s * wb_ref[...], axis=1, keepdims=True)
    bta = jax.nn.sigmoid(z)
    y = bta * s + (1.0 - bta) * o
    c_ref[...] = jnp.where(y > 0, y, jnp.exp(jnp.minimum(y, 0.0)) - 1.0)


def _gate(out, skip, p):
    tb = p['tbeta_w']
    wa = (tb[0:128, 0] + tb[256:384, 0]).reshape(1, H)
    wb = (tb[128:256, 0] - tb[256:384, 0]).reshape(1, H)
    return pl.pallas_call(
        _gate_body,
        grid=(N // _RT,),
        in_specs=[
            pl.BlockSpec((_RT, H), lambda i: (i, 0)),
            pl.BlockSpec((_RT, H), lambda i: (i, 0)),
            pl.BlockSpec((1, H), lambda i: (0, 0)),
            pl.BlockSpec((1, H), lambda i: (0, 0)),
        ],
        out_specs=pl.BlockSpec((_RT, H), lambda i: (i, 0)),
        out_shape=jax.ShapeDtypeStruct((N, H), jnp.float32),
        interpret=_INTERPRET,
    )(out, skip, wa, wb)


_R = 256
_CW = 512
_BIG = 1e30
_INF = np.float32(np.inf)


def _knn_body(batch_sm, starts_sm, x_ref, xt_ref, bcol_ref, sqcol_ref,
              brow_ref, out_ref):
    i = pl.program_id(0)
    xi = x_ref[...]
    bi = brow_ref[...]
    r0 = i * _R
    b_lo = jnp.clip(batch_sm[jnp.minimum(r0, N - 1)], 0, G - 1)
    b_hi = jnp.clip(batch_sm[jnp.minimum(r0 + _R - 1, N - 1)], 0, G - 1)
    jstart = starts_sm[b_lo]
    jend = starts_sm[b_hi + 1]
    j0 = jstart // _CW
    j1 = (jend + _CW - 1) // _CW

    row_ids = r0 + jax.lax.broadcasted_iota(jnp.int32, (_R, 1), 0)
    lane128 = jax.lax.broadcasted_iota(jnp.int32, (_R, 128), 1)
    run_v = jnp.where(lane128 < K, jnp.float32(_BIG), _INF)
    run_i = jnp.where(lane128 < K, lane128, 0)

    W = 128 + _CW
    lane = jax.lax.broadcasted_iota(jnp.int32, (_R, W), 1)

    def body(j, carry):
        run_v, run_i = carry
        c0 = pl.multiple_of(j * _CW, _CW)
        xj = xt_ref[:, pl.ds(c0, _CW)]
        prod = jax.lax.dot_general(
            xi, xj, (((1,), (0,)), ((), ())),
            preferred_element_type=jnp.float32)
        sqj = sqcol_ref[:, pl.ds(c0, _CW)]
        bj = bcol_ref[:, pl.ds(c0, _CW)]
        col_ids = c0 + jax.lax.broadcasted_iota(jnp.int32, (1, _CW), 1)
        d2 = sqj - 2.0 * prod
        d2 = jnp.where(bi != bj, jnp.float32(_BIG), d2)
        d2 = jnp.where(row_ids == col_ids, jnp.float32(_BIG), d2)
        cand_v = jnp.concatenate([run_v, d2], axis=1)
        cand_i = jnp.concatenate(
            [run_i, jnp.broadcast_to(col_ids, (_R, _CW))], axis=1)
        new_v, new_i = [], []
        for _ in range(K):
            m = jnp.min(cand_v, axis=1, keepdims=True)
            posm = jnp.where(cand_v == m, lane, W)
            pos = jnp.min(posm, axis=1, keepdims=True)
            sel = lane == pos
            idxk = jnp.max(jnp.where(sel, cand_i, -1), axis=1, keepdims=True)
            new_v.append(m)
            new_i.append(idxk)
            cand_v = jnp.where(sel, _INF, cand_v)
        pad_v = jnp.full((_R, 128 - K), _INF, jnp.float32)
        pad_i = jnp.zeros((_R, 128 - K), jnp.int32)
        return (jnp.concatenate(new_v + [pad_v], axis=1),
                jnp.concatenate(new_i + [pad_i], axis=1))

    run_v, run_i = jax.lax.fori_loop(j0, j1, body, (run_v, run_i))
    out_ref[...] = run_i


def _knn_pallas(x, batch, starts):
    xp = jnp.zeros((_NP, D), jnp.float32).at[:N].set(x)
    xt = jnp.zeros((D, _NP), jnp.float32).at[:, :N].set(x.T)
    bcol = jnp.full((1, _NP), -1, jnp.int32).at[0, :N].set(batch)
    brow = jnp.full((_NP, 1), G, jnp.int32).at[:N, 0].set(batch)
    sqcol = jnp.zeros((1, _NP), jnp.float32).at[0, :N].set(
        jnp.sum(x * x, axis=1))
    grid_spec = pltpu.PrefetchScalarGridSpec(
        num_scalar_prefetch=2,
        grid=(_NP // _R,),
        in_specs=[
            pl.BlockSpec((_R, D), lambda i, *_: (i, 0)),
            pl.BlockSpec((D, _NP), lambda i, *_: (0, 0)),
            pl.BlockSpec((1, _NP), lambda i, *_: (0, 0)),
            pl.BlockSpec((1, _NP), lambda i, *_: (0, 0)),
            pl.BlockSpec((_R, 1), lambda i, *_: (i, 0)),
        ],
        out_specs=pl.BlockSpec((_R, 128), lambda i, *_: (i, 0)),
    )
    out = pl.pallas_call(
        _knn_body,
        grid_spec=grid_spec,
        out_shape=jax.ShapeDtypeStruct((_NP, 128), jnp.int32),
        interpret=_INTERPRET,
    )(batch, starts, xp, xt, bcol, sqcol, brow)
    return out[:N, :K]


def _ab_body(x_ref, wa_ref, ba_ref, wb_ref, wsc_ref, bsc_ref,
             a_ref, b_ref, sc_ref):
    x = x_ref[...]
    a_ref[...] = jax.lax.dot_general(
        x, wa_ref[...], (((1,), (0,)), ((), ())),
        preferred_element_type=jnp.float32) + ba_ref[...]
    b_ref[...] = jax.lax.dot_general(
        x, wb_ref[...], (((1,), (0,)), ((), ())),
        preferred_element_type=jnp.float32)
    sc_ref[...] = jax.lax.dot_general(
        x, wsc_ref[...], (((1,), (0,)), ((), ())),
        preferred_element_type=jnp.float32) + bsc_ref[...]


def _ab_sc(x, wa, ba, wb, wsc, bsc):
    return pl.pallas_call(
        _ab_body,
        grid=(N // _RT,),
        in_specs=[pl.BlockSpec((_RT, D), lambda i: (i, 0))] +
                 [pl.BlockSpec((D, D), lambda i: (0, 0)),
                  pl.BlockSpec((1, D), lambda i: (0, 0)),
                  pl.BlockSpec((D, D), lambda i: (0, 0)),
                  pl.BlockSpec((D, D), lambda i: (0, 0)),
                  pl.BlockSpec((1, D), lambda i: (0, 0))],
        out_specs=tuple(pl.BlockSpec((_RT, D), lambda i: (i, 0))
                        for _ in range(3)),
        out_shape=tuple(jax.ShapeDtypeStruct((N, D), jnp.float32)
                        for _ in range(3)),
        interpret=_INTERPRET,
    )(x, wa, ba, wb, wsc, bsc)


_NWK = 32
_RPW = _NP // _NWK


_GCH = 128
_C0_SHARE_PCT = 30


def _sc_gather(idx, table):
    M = idx.shape[0]
    Wd = table.shape[1]
    rpw = M // _NWK
    nch = rpw // _GCH
    nb = 4 if (nch % 4 == 0 and Wd <= 128) else 2
    w0 = max(nb, ((2 * nch * _C0_SHARE_PCT // 100) // nb) * nb)

    def body(idx_hbm, table_hbm, out_hbm, idx_v, *rows_sem):
        rows, sem = rows_sem[:nb], rows_sem[nb]
        sid = lax.axis_index("s")
        cid = lax.axis_index("c")
        pair_first = sid * 2 * nch
        my_cnt = jnp.where(cid == 0, w0, 2 * nch - w0)
        my_loc = jnp.where(cid == 0, 0, w0)
        pltpu.sync_copy(idx_hbm.at[pl.ds(pair_first * _GCH, 2 * rpw)], idx_v)

        def macro(m):
            cps = []
            for b in range(nb):
                loc = (my_loc + m * nb + b) * _GCH
                cps.append(pltpu.async_copy(
                    table_hbm.at[idx_v.at[pl.ds(loc, _GCH)]],
                    rows[b], sem))
            for cp in cps:
                cp.wait()
            for b in range(nb):
                loc = (my_loc + m * nb + b) * _GCH
                pltpu.sync_copy(rows[b],
                                out_hbm.at[pl.ds(pair_first * _GCH + loc,
                                                 _GCH)])

        pl.loop(0, my_cnt // nb)(macro)

    mesh = plsc.VectorSubcoreMesh(core_axis_name="c", subcore_axis_name="s")
    f = pl.kernel(
        body,
        mesh=mesh,
        out_type=jax.ShapeDtypeStruct((M, Wd), jnp.float32),
        scratch_types=[pltpu.VMEM((2 * rpw,), jnp.int32)] +
                      [pltpu.VMEM((_GCH, Wd), jnp.float32)
                       for _ in range(nb)] +
                      [pltpu.SemaphoreType.DMA],
    )
    return f(idx, table)


def _econv_body(a_ref, bg_ref, sc_ref, w2_ref, b2_ref, w3_ref, b3_ref,
                s3_ref, t3_ref, out_ref):
    a = a_ref[...]
    acc = None
    for k in range(K):
        h = jnp.maximum(a + bg_ref[k], 0.0)
        h = jnp.maximum(jax.lax.dot_general(
            h, w2_ref[...], (((1,), (0,)), ((), ())),
            preferred_element_type=jnp.float32) + b2_ref[...], 0.0)
        h = jnp.maximum(jax.lax.dot_general(
            h, w3_ref[...], (((1,), (0,)), ((), ())),
            preferred_element_type=jnp.float32) + b3_ref[...], 0.0)
        h = h * s3_ref[...] + t3_ref[...]
        acc = h if acc is None else jnp.maximum(acc, h)
    out_ref[...] = acc + sc_ref[...]


def _econv(a, bg, sc, w2, b2, w3, b3, s3, t3):
    return pl.pallas_call(
        _econv_body,
        grid=(N // _RT,),
        in_specs=[
            pl.BlockSpec((_RT, D), lambda i: (i, 0)),
            pl.BlockSpec((K, _RT, D), lambda i: (0, i, 0)),
            pl.BlockSpec((_RT, D), lambda i: (i, 0)),
            pl.BlockSpec((D, D), lambda i: (0, 0)),
            pl.BlockSpec((1, D), lambda i: (0, 0)),
            pl.BlockSpec((D, D), lambda i: (0, 0)),
            pl.BlockSpec((1, D), lambda i: (0, 0)),
            pl.BlockSpec((1, D), lambda i: (0, 0)),
            pl.BlockSpec((1, D), lambda i: (0, 0)),
        ],
        out_specs=pl.BlockSpec((_RT, D), lambda i: (i, 0)),
        out_shape=jax.ShapeDtypeStruct((N, D), jnp.float32),
        interpret=_INTERPRET,
    )(a, bg, sc, w2, b2, w3, b3, s3, t3)


def _bn_fold(w, b):
    s = w / np.sqrt(1.0 + EPS)
    return s, b


def _dyn_edge_conv(x, batch, starts, p, c):
    nbr = _knn_pallas(x, batch, starts)
    s1, t1 = _bn_fold(p[c + 'bn1_w'], p[c + 'bn1_b'])
    s2, t2 = _bn_fold(p[c + 'bn2_w'], p[c + 'bn2_b'])
    s3, t3 = _bn_fold(p[c + 'bn3_w'], p[c + 'bn3_b'])
    ssc, tsc = _bn_fold(p[c + 'scbn_w'], p[c + 'scbn_b'])
    w1a = p[c + 'l1_w'][:D]
    w1b = p[c + 'l1_w'][D:]
    wa = w1a - w1b
    ba = p[c + 'l1_b'].reshape(1, H)
    w2f = s1[:, None] * p[c + 'l2_w']
    b2f = (t1 @ p[c + 'l2_w'] + p[c + 'l2_b']).reshape(1, H)
    w3f = s2[:, None] * p[c + 'l3_w']
    b3f = (t2 @ p[c + 'l3_w'] + p[c + 'l3_b']).reshape(1, H)
    wscf = p[c + 'sc_w'] * ssc[None, :]
    bscf = (p[c + 'sc_b'] * ssc + tsc).reshape(1, H)

    a, b, sc = _ab_sc(x, wa, ba, w1b, wscf, bscf)
    nbrT = jnp.zeros((K, _NP), jnp.int32).at[:, :N].set(nbr.T).reshape(-1)
    bg = _sc_gather(nbrT, b).reshape(K, _NP, D)
    return _econv(a, bg, sc, w2f, b2f, w3f, b3f,
                  s3.reshape(1, H), t3.reshape(1, H))


def _head_body(c1_ref, c2_ref, c3_ref, bcol_ref, gi_ref, w1g_ref, w1i_ref,
               b1_ref, w2_ref, b2_ref, wo_ref, bo_ref, out_ref):
    s = c1_ref[...] + c2_ref[...] + c3_ref[...]
    g_col = jax.lax.broadcasted_iota(jnp.int32, (G, 1), 0)
    ohT = (g_col == bcol_ref[...]).astype(jnp.float32)
    cnt = jnp.maximum(jnp.sum(ohT, axis=1, keepdims=True), 1.0)
    g = jax.lax.dot_general(ohT, s, (((1,), (0,)), ((), ())),
                            preferred_element_type=jnp.float32) / cnt
    g1 = jax.lax.dot_general(g, w1g_ref[...], (((1,), (0,)), ((), ())),
                             preferred_element_type=jnp.float32)
    g1 = g1 + jax.lax.dot_general(gi_ref[...], w1i_ref[...],
                                  (((1,), (0,)), ((), ())),
                                  preferred_element_type=jnp.float32)
    g1 = g1 + b1_ref[...]
    alpha = 1.6732632423543772
    scale = 1.0507009873554805
    g1 = scale * jnp.where(g1 > 0, g1,
                           alpha * (jnp.exp(jnp.minimum(g1, 0.0)) - 1.0))
    g2 = jax.lax.dot_general(g1, w2_ref[...], (((1,), (0,)), ((), ())),
                             preferred_element_type=jnp.float32) + b2_ref[...]
    g2 = scale * jnp.where(g2 > 0, g2,
                           alpha * (jnp.exp(jnp.minimum(g2, 0.0)) - 1.0))
    logits = jax.lax.dot_general(g2, wo_ref[...], (((1,), (0,)), ((), ())),
                                 preferred_element_type=jnp.float32) + bo_ref[...]
    lane = jax.lax.broadcasted_iota(jnp.int32, (G, 128), 1)
    ml = lane < C
    lm = jnp.where(ml, logits, -_INF)
    m = jnp.max(lm, axis=1, keepdims=True)
    ex = jnp.where(ml, jnp.exp(lm - m), 0.0)
    lse = jnp.log(jnp.sum(ex, axis=1, keepdims=True))
    out_ref[...] = logits - m - lse


def _head(c1, c2, c3, batch, graph_input, p):
    s0, t0 = _bn_fold(p['bn0_w'], p['bn0_b'])
    s0a, s0b = s0[:H], s0[H:]
    t0a, t0b = t0[:H], t0[H:]
    w1g = s0a[:, None] * p['d1_w'][:H]
    w1i = s0b[:, None] * p['d1_w'][H:]
    b1 = (t0a @ p['d1_w'][:H] + t0b @ p['d1_w'][H:] + p['d1_b']).reshape(1, H)
    w1i_p = jnp.zeros((128, H), jnp.float32).at[:GF].set(w1i)
    gi_p = jnp.zeros((G, 128), jnp.float32).at[:, :GF].set(graph_input)
    wo_p = jnp.zeros((H, 128), jnp.float32).at[:, :C].set(p['out_w'])
    bo_p = jnp.zeros((1, 128), jnp.float32).at[0, :C].set(p['out_b'])
    out = pl.pallas_call(
        _head_body,
        out_shape=jax.ShapeDtypeStruct((G, 128), jnp.float32),
        interpret=_INTERPRET,
    )(c1, c2, c3, batch.reshape(1, N), gi_p, w1g, w1i_p, b1,
      p['d2_w'], p['d2_b'].reshape(1, H), wo_p, bo_p)
    return out[:, :C]


_EP = 163840
_EPW = _EP // _NWK
_ECH = 128
_NCH = _EPW // _ECH
_ET = 2048
_AW = 144
_ISQH = float(1.0 / np.sqrt(float(H)))
E_REAL = 160000


def _edot_body(qd_ref, ks_ref, a_ref):
    a_ref[...] = jnp.sum(qd_ref[...] * ks_ref[...], axis=1, keepdims=True)


def _edot(qd, kvg):
    return pl.pallas_call(
        _edot_body,
        grid=(_EP // _ET,),
        in_specs=[pl.BlockSpec((_ET, H), lambda i: (i, 0)),
                  pl.BlockSpec((_ET, H), lambda i: (i, 0))],
        out_specs=pl.BlockSpec((_ET, 1), lambda i: (i, 0)),
        out_shape=jax.ShapeDtypeStruct((_EP, 1), jnp.float32),
        interpret=_INTERPRET,
    )(qd, kvg)


def _gmax_body(a_ref, out_ref):
    out_ref[...] = jnp.full((1, 128), jnp.max(a_ref[...]), jnp.float32)


def _gmax(a2d):
    return pl.pallas_call(
        _gmax_body,
        out_shape=jax.ShapeDtypeStruct((1, 128), jnp.float32),
        interpret=_INTERPRET,
    )(a2d)


def _st_body(a_ref, vs_ref, dm_ref, gm_ref, stn_ref, std_ref):
    i = pl.program_id(0)
    a = a_ref[...]
    ex = jnp.exp(a * _ISQH - gm_ref[0:1, 0:1])
    gid = i * _ET + jax.lax.broadcasted_iota(jnp.int32, (_ET, 1), 0)
    ex = jnp.where(gid < E_REAL, ex, 0.0)
    stn_ref[...] = ex * vs_ref[...]
    lane = jax.lax.broadcasted_iota(jnp.int32, (_ET, 128), 1)
    std_ref[...] = jnp.where(lane == dm_ref[...], ex, 0.0)


def _st_build(a, kvg, dmod, gm):
    return pl.pallas_call(
        _st_body,
        grid=(_EP // _ET,),
        in_specs=[pl.BlockSpec((_ET, 1), lambda i: (i, 0)),
                  pl.BlockSpec((_ET, H), lambda i: (i, 1)),
                  pl.BlockSpec((_ET, 1), lambda i: (i, 0)),
                  pl.BlockSpec((1, 128), lambda i: (0, 0))],
        out_specs=(pl.BlockSpec((_ET, H), lambda i: (i, 0)),
                   pl.BlockSpec((_ET, 128), lambda i: (i, 0))),
        out_shape=(jax.ShapeDtypeStruct((_EP, H), jnp.float32),
                   jax.ShapeDtypeStruct((_EP, 128), jnp.float32)),
        interpret=_INTERPRET,
    )(a, kvg, dmod, gm)


_DDIV = 128


def _sc_scatter_add(st, idx3, acc_rows):
    slab = acc_rows // 16

    def body(st_hbm, idx_hbm, z_hbm, acc_out, idxv, st0, st1, acc_sh, sem):
        cid = lax.axis_index("c")
        tid = lax.axis_index("s")
        wid = tid * 2 + cid
        base = wid * _EPW

        pltpu.sync_copy(z_hbm.at[pl.ds(0, slab)],
                        acc_sh.at[pl.ds(tid * slab, slab)])
        pltpu.sync_copy(idx_hbm.at[wid], idxv)
        plsc.subcore_barrier()

        sts = (st0, st1)

        def macro(m):
            cps = []
            for b in range(2):
                off = base + (m * 2 + b) * _ECH
                cps.append(pltpu.async_copy(st_hbm.at[pl.ds(off, _ECH)],
                                            sts[b], sem))
            cps[0].wait()
            pltpu.sync_copy(sts[0], acc_sh.at[idxv.at[m * 2]], add=True)
            cps[1].wait()
            pltpu.sync_copy(sts[1], acc_sh.at[idxv.at[m * 2 + 1]], add=True)

        pl.loop(0, _NCH // 2)(macro)
        plsc.subcore_barrier()
        pltpu.sync_copy(acc_sh.at[pl.ds(tid * slab, slab)],
                        acc_out.at[pl.ds(cid * acc_rows + tid * slab, slab)])

    z = jnp.zeros((max(_NP // 16, 128), 128), jnp.float32)
    mesh = plsc.VectorSubcoreMesh(core_axis_name="c", subcore_axis_name="s")
    f = pl.kernel(
        body,
        mesh=mesh,
        out_type=jax.ShapeDtypeStruct((2 * acc_rows, 128), jnp.float32),
        scratch_types=[
            pltpu.VMEM((_NCH, _ECH), jnp.int32),
            pltpu.VMEM((_ECH, 128), jnp.float32),
            pltpu.VMEM((_ECH, 128), jnp.float32),
            pltpu.VMEM_SHARED((acc_rows, 128), jnp.float32),
            pltpu.SemaphoreType.DMA,
        ],
    )
    return f(st, idx3, z)


def _attention_sc(q, kv, src, dst):
    srcp = jnp.zeros((_EP,), jnp.int32).at[:E_REAL].set(src)
    dstp = jnp.zeros((_EP,), jnp.int32).at[:E_REAL].set(dst)
    qd = _sc_gather(dstp, q)
    kvg = _sc_gather(srcp, kv)
    a = _edot(qd, kvg)
    gm = _gmax(a.reshape(_EP // 128, 128))
    stn, std = _st_build(a, kvg, (dstp % 128).reshape(_EP, 1), gm)
    accn = _sc_scatter_add(stn, dstp.reshape(_NWK, _NCH, _ECH), _NP)
    accd = _sc_scatter_add(std, (dstp // 128).reshape(_NWK, _NCH, _ECH),
                           _DDIV)
    num = accn.reshape(2, _NP, 128)
    den = (accd[:_DDIV] + accd[_DDIV:]).reshape(-1)[:_NP].reshape(_NP, 1)
    return num, den


def _gate_acc_body(acc_ref, den_ref, s_ref, wa_ref, wb_ref, c_ref):
    num = acc_ref[0] + acc_ref[1]
    den = den_ref[...]
    o = num / jnp.maximum(den, 1e-30)
    s = s_ref[...]
    z = jnp.sum(o * wa_ref[...] + s * wb_ref[...], axis=1, keepdims=True)
    bta = jax.nn.sigmoid(z)
    y = bta * s + (1.0 - bta) * o
    c_ref[...] = jnp.where(y > 0, y, jnp.exp(jnp.minimum(y, 0.0)) - 1.0)


def _gate_acc(num, den, skip, p):
    tb = p['tbeta_w']
    wa = (tb[0:128, 0] + tb[256:384, 0]).reshape(1, H)
    wb = (tb[128:256, 0] - tb[256:384, 0]).reshape(1, H)
    return pl.pallas_call(
        _gate_acc_body,
        grid=(N // _RT,),
        in_specs=[
            pl.BlockSpec((2, _RT, 128), lambda i: (0, i, 0)),
            pl.BlockSpec((_RT, 1), lambda i: (i, 0)),
            pl.BlockSpec((_RT, H), lambda i: (i, 0)),
            pl.BlockSpec((1, H), lambda i: (0, 0)),
            pl.BlockSpec((1, H), lambda i: (0, 0)),
        ],
        out_specs=pl.BlockSpec((_RT, H), lambda i: (i, 0)),
        out_shape=jax.ShapeDtypeStruct((N, H), jnp.float32),
        interpret=_INTERPRET,
    )(num, den, skip, wa, wb)


def kernel(x, edge_index, graph_input, batch, params):
    src, dst = edge_index[0], edge_index[1]
    h, starts = _graph_norm_starts(x, batch, params)
    q, kv, skip = _projections(h, params)
    num, den = _attention_sc(q, kv, src, dst)
    c1 = _gate_acc(num, den, skip, params)
    c2 = _dyn_edge_conv(c1, batch, starts, params, 'c2_')
    c3 = _dyn_edge_conv(c2, batch, starts, params, 'c3_')
    return _head(c1, c2, c3, batch, graph_input, params)


# speedup vs baseline: 1.0817x; 1.0817x over previous
import functools
from functools import partial

import jax
import jax.numpy as jnp
import numpy as np
from jax import lax
from jax.experimental import pallas as pl
from jax.experimental.pallas import tpu as pltpu
from jax.experimental.pallas import tpu_sc as plsc

N = 10000
D = 128
G = 64
GF = 16
H = 128
C = 10
K = 4
EPS = 1e-5

_INTERPRET = False

_NP = 10240
_RT = 1000


def _norm_body(x_ref, brow_ref, bcol_ref, ms_ref, w_ref, b_ref,
               h_ref, starts_ref):
    x = x_ref[...]
    brow = brow_ref[...]
    bcol = bcol_ref[...]
    g_row = jax.lax.broadcasted_iota(jnp.int32, (1, 128), 1)
    g_col = jax.lax.broadcasted_iota(jnp.int32, (128, 1), 0)
    oh = (brow == g_row).astype(jnp.float32)
    ohT = (g_col == bcol).astype(jnp.float32)
    cnt = jnp.sum(ohT, axis=1, keepdims=True)
    cnt = jnp.maximum(cnt, 1.0)
    sums = jax.lax.dot_general(ohT, x, (((1,), (0,)), ((), ())),
                               preferred_element_type=jnp.float32)
    mean = sums / cnt
    meanb = jax.lax.dot_general(oh, mean, (((1,), (0,)), ((), ())),
                                preferred_element_type=jnp.float32)
    xc = x - ms_ref[...] * meanb
    var = jax.lax.dot_general(ohT, xc * xc, (((1,), (0,)), ((), ())),
                              preferred_element_type=jnp.float32) / cnt
    std = jnp.sqrt(var + EPS)
    stdb = jax.lax.dot_general(oh, std, (((1,), (0,)), ((), ())),
                               preferred_element_type=jnp.float32)
    h_ref[...] = w_ref[...] * xc / stdb + b_ref[...]
    tri = (g_col < g_row).astype(jnp.float32)
    cnt_row = jnp.sum(oh, axis=0, keepdims=True)
    starts_f = jax.lax.dot_general(cnt_row, tri, (((1,), (0,)), ((), ())),
                                   preferred_element_type=jnp.float32)
    starts_ref[...] = starts_f.astype(jnp.int32)


def _graph_norm_starts(x, batch, p):
    brow = batch.reshape(N, 1)
    bcol = batch.reshape(1, N)
    ms = p['gn_ms'].reshape(1, D)
    w = p['gn_w'].reshape(1, D)
    b = p['gn_b'].reshape(1, D)
    h, starts = pl.pallas_call(
        _norm_body,
        out_shape=(jax.ShapeDtypeStruct((N, D), jnp.float32),
                   jax.ShapeDtypeStruct((1, 128), jnp.int32)),
        interpret=_INTERPRET,
    )(x, brow, bcol, ms, w, b)
    return h, starts[0, :G + 1]


def _proj_body(x_ref, w_ref, b_ref, q_ref, kv_ref, s_ref):
    y = jax.lax.dot_general(x_ref[...], w_ref[...], (((1,), (0,)), ((), ())),
                            preferred_element_type=jnp.float32) + b_ref[...]
    q_ref[...] = y[:, 0:128]
    kv_ref[...] = y[:, 128:384]
    s_ref[...] = y[:, 384:512]


def _projections(h, p):
    w4 = jnp.concatenate([p['tq_w'], p['tk_w'], p['tv_w'], p['ts_w']], axis=1)
    b4 = jnp.concatenate([p['tq_b'], p['tk_b'], p['tv_b'], p['ts_b']]
                         ).reshape(1, 512)
    outs = pl.pallas_call(
        _proj_body,
        grid=(N // _RT,),
        in_specs=[
            pl.BlockSpec((_RT, D), lambda i: (i, 0)),
            pl.BlockSpec((D, 512), lambda i: (0, 0)),
            pl.BlockSpec((1, 512), lambda i: (0, 0)),
        ],
        out_specs=(pl.BlockSpec((_RT, H), lambda i: (i, 0)),
                   pl.BlockSpec((_RT, 256), lambda i: (i, 0)),
                   pl.BlockSpec((_RT, H), lambda i: (i, 0))),
        out_shape=(jax.ShapeDtypeStruct((N, H), jnp.float32),
                   jax.ShapeDtypeStruct((N, 256), jnp.float32),
                   jax.ShapeDtypeStruct((N, H), jnp.float32)),
        interpret=_INTERPRET,
    )(h, w4, b4)
    return outs


def _gate_body(o_ref, s_ref, wa_ref, wb_ref, c_ref):
    o = o_ref[...]
    s = s_ref[...]
    z = jnp.sum(o * wa_ref[...] + s * wb_ref[...], axis=1, keepdims=True)
    bta = jax.nn.sigmoid(z)
    y = bta * s + (1.0 - bta) * o
    c_ref[...] = jnp.where(y > 0, y, jnp.exp(jnp.minimum(y, 0.0)) - 1.0)


def _gate(out, skip, p):
    tb = p['tbeta_w']
    wa = (tb[0:128, 0] + tb[256:384, 0]).reshape(1, H)
    wb = (tb[128:256, 0] - tb[256:384, 0]).reshape(1, H)
    return pl.pallas_call(
        _gate_body,
        grid=(N // _RT,),
        in_specs=[
            pl.BlockSpec((_RT, H), lambda i: (i, 0)),
            pl.BlockSpec((_RT, H), lambda i: (i, 0)),
            pl.BlockSpec((1, H), lambda i: (0, 0)),
            pl.BlockSpec((1, H), lambda i: (0, 0)),
        ],
        out_specs=pl.BlockSpec((_RT, H), lambda i: (i, 0)),
        out_shape=jax.ShapeDtypeStruct((N, H), jnp.float32),
        interpret=_INTERPRET,
    )(out, skip, wa, wb)


_R = 256
_CW = 512
_BIG = 1e30
_INF = np.float32(np.inf)


def _knn_body(batch_sm, starts_sm, x_ref, xt_ref, bcol_ref, sqcol_ref,
              brow_ref, out_ref):
    i = pl.program_id(0)
    xi = x_ref[...]
    bi = brow_ref[...]
    r0 = i * _R
    b_lo = jnp.clip(batch_sm[jnp.minimum(r0, N - 1)], 0, G - 1)
    b_hi = jnp.clip(batch_sm[jnp.minimum(r0 + _R - 1, N - 1)], 0, G - 1)
    jstart = starts_sm[b_lo]
    jend = starts_sm[b_hi + 1]
    j0 = jstart // _CW
    j1 = (jend + _CW - 1) // _CW

    row_ids = r0 + jax.lax.broadcasted_iota(jnp.int32, (_R, 1), 0)
    lane128 = jax.lax.broadcasted_iota(jnp.int32, (_R, 128), 1)
    run_v = jnp.where(lane128 < K, jnp.float32(_BIG), _INF)
    run_i = jnp.where(lane128 < K, lane128, 0)

    W = 128 + _CW
    lane = jax.lax.broadcasted_iota(jnp.int32, (_R, W), 1)

    def body(j, carry):
        run_v, run_i = carry
        c0 = pl.multiple_of(j * _CW, _CW)
        xj = xt_ref[:, pl.ds(c0, _CW)]
        prod = jax.lax.dot_general(
            xi, xj, (((1,), (0,)), ((), ())),
            preferred_element_type=jnp.float32)
        sqj = sqcol_ref[:, pl.ds(c0, _CW)]
        bj = bcol_ref[:, pl.ds(c0, _CW)]
        col_ids = c0 + jax.lax.broadcasted_iota(jnp.int32, (1, _CW), 1)
        d2 = sqj - 2.0 * prod
        d2 = jnp.where(bi != bj, jnp.float32(_BIG), d2)
        d2 = jnp.where(row_ids == col_ids, jnp.float32(_BIG), d2)
        cand_v = jnp.concatenate([run_v, d2], axis=1)
        cand_i = jnp.concatenate(
            [run_i, jnp.broadcast_to(col_ids, (_R, _CW))], axis=1)
        new_v, new_i = [], []
        for _ in range(K):
            m = jnp.min(cand_v, axis=1, keepdims=True)
            posm = jnp.where(cand_v == m, lane, W)
            pos = jnp.min(posm, axis=1, keepdims=True)
            sel = lane == pos
            idxk = jnp.max(jnp.where(sel, cand_i, -1), axis=1, keepdims=True)
            new_v.append(m)
            new_i.append(idxk)
            cand_v = jnp.where(sel, _INF, cand_v)
        pad_v = jnp.full((_R, 128 - K), _INF, jnp.float32)
        pad_i = jnp.zeros((_R, 128 - K), jnp.int32)
        return (jnp.concatenate(new_v + [pad_v], axis=1),
                jnp.concatenate(new_i + [pad_i], axis=1))

    run_v, run_i = jax.lax.fori_loop(j0, j1, body, (run_v, run_i))
    out_ref[...] = run_i


def _knn_pallas(x, batch, starts):
    xp = jnp.zeros((_NP, D), jnp.float32).at[:N].set(x)
    xt = jnp.zeros((D, _NP), jnp.float32).at[:, :N].set(x.T)
    bcol = jnp.full((1, _NP), -1, jnp.int32).at[0, :N].set(batch)
    brow = jnp.full((_NP, 1), G, jnp.int32).at[:N, 0].set(batch)
    sqcol = jnp.zeros((1, _NP), jnp.float32).at[0, :N].set(
        jnp.sum(x * x, axis=1))
    grid_spec = pltpu.PrefetchScalarGridSpec(
        num_scalar_prefetch=2,
        grid=(_NP // _R,),
        in_specs=[
            pl.BlockSpec((_R, D), lambda i, *_: (i, 0)),
            pl.BlockSpec((D, _NP), lambda i, *_: (0, 0)),
            pl.BlockSpec((1, _NP), lambda i, *_: (0, 0)),
            pl.BlockSpec((1, _NP), lambda i, *_: (0, 0)),
            pl.BlockSpec((_R, 1), lambda i, *_: (i, 0)),
        ],
        out_specs=pl.BlockSpec((_R, 128), lambda i, *_: (i, 0)),
    )
    out = pl.pallas_call(
        _knn_body,
        grid_spec=grid_spec,
        out_shape=jax.ShapeDtypeStruct((_NP, 128), jnp.int32),
        interpret=_INTERPRET,
    )(batch, starts, xp, xt, bcol, sqcol, brow)
    return out[:N, :K]


def _ab_body(x_ref, wa_ref, ba_ref, wb_ref, wsc_ref, bsc_ref,
             a_ref, b_ref, sc_ref):
    x = x_ref[...]
    a_ref[...] = jax.lax.dot_general(
        x, wa_ref[...], (((1,), (0,)), ((), ())),
        preferred_element_type=jnp.float32) + ba_ref[...]
    b_ref[...] = jax.lax.dot_general(
        x, wb_ref[...], (((1,), (0,)), ((), ())),
        preferred_element_type=jnp.float32)
    sc_ref[...] = jax.lax.dot_general(
        x, wsc_ref[...], (((1,), (0,)), ((), ())),
        preferred_element_type=jnp.float32) + bsc_ref[...]


def _ab_sc(x, wa, ba, wb, wsc, bsc):
    return pl.pallas_call(
        _ab_body,
        grid=(N // _RT,),
        in_specs=[pl.BlockSpec((_RT, D), lambda i: (i, 0))] +
                 [pl.BlockSpec((D, D), lambda i: (0, 0)),
                  pl.BlockSpec((1, D), lambda i: (0, 0)),
                  pl.BlockSpec((D, D), lambda i: (0, 0)),
                  pl.BlockSpec((D, D), lambda i: (0, 0)),
                  pl.BlockSpec((1, D), lambda i: (0, 0))],
        out_specs=tuple(pl.BlockSpec((_RT, D), lambda i: (i, 0))
                        for _ in range(3)),
        out_shape=tuple(jax.ShapeDtypeStruct((N, D), jnp.float32)
                        for _ in range(3)),
        interpret=_INTERPRET,
    )(x, wa, ba, wb, wsc, bsc)


_NWK = 32
_RPW = _NP // _NWK


_GCH = 128
_C0_SHARE_PCT = 70


def _sc_gather(idx, table):
    M = idx.shape[0]
    Wd = table.shape[1]
    rpw = M // _NWK
    nch = rpw // _GCH
    nb = 4 if (nch % 4 == 0 and Wd <= 128) else 2
    w0 = max(nb, ((2 * nch * _C0_SHARE_PCT // 100) // nb) * nb)

    def body(idx_hbm, table_hbm, out_hbm, idx_v, *rows_sem):
        rows, sem = rows_sem[:nb], rows_sem[nb]
        sid = lax.axis_index("s")
        cid = lax.axis_index("c")
        pair_first = sid * 2 * nch
        my_cnt = jnp.where(cid == 0, w0, 2 * nch - w0)
        my_loc = jnp.where(cid == 0, 0, w0)
        pltpu.sync_copy(idx_hbm.at[pl.ds(pair_first * _GCH, 2 * rpw)], idx_v)

        def macro(m):
            cps = []
            for b in range(nb):
                loc = (my_loc + m * nb + b) * _GCH
                cps.append(pltpu.async_copy(
                    table_hbm.at[idx_v.at[pl.ds(loc, _GCH)]],
                    rows[b], sem))
            for cp in cps:
                cp.wait()
            for b in range(nb):
                loc = (my_loc + m * nb + b) * _GCH
                pltpu.sync_copy(rows[b],
                                out_hbm.at[pl.ds(pair_first * _GCH + loc,
                                                 _GCH)])

        pl.loop(0, my_cnt // nb)(macro)

    mesh = plsc.VectorSubcoreMesh(core_axis_name="c", subcore_axis_name="s")
    f = pl.kernel(
        body,
        mesh=mesh,
        out_type=jax.ShapeDtypeStruct((M, Wd), jnp.float32),
        scratch_types=[pltpu.VMEM((2 * rpw,), jnp.int32)] +
                      [pltpu.VMEM((_GCH, Wd), jnp.float32)
                       for _ in range(nb)] +
                      [pltpu.SemaphoreType.DMA],
    )
    return f(idx, table)


def _econv_body(a_ref, bg_ref, sc_ref, w2_ref, b2_ref, w3_ref, b3_ref,
                s3_ref, t3_ref, out_ref):
    a = a_ref[...]
    acc = None
    for k in range(K):
        h = jnp.maximum(a + bg_ref[k], 0.0)
        h = jnp.maximum(jax.lax.dot_general(
            h, w2_ref[...], (((1,), (0,)), ((), ())),
            preferred_element_type=jnp.float32) + b2_ref[...], 0.0)
        h = jnp.maximum(jax.lax.dot_general(
            h, w3_ref[...], (((1,), (0,)), ((), ())),
            preferred_element_type=jnp.float32) + b3_ref[...], 0.0)
        h = h * s3_ref[...] + t3_ref[...]
        acc = h if acc is None else jnp.maximum(acc, h)
    out_ref[...] = acc + sc_ref[...]


def _econv(a, bg, sc, w2, b2, w3, b3, s3, t3):
    return pl.pallas_call(
        _econv_body,
        grid=(N // _RT,),
        in_specs=[
            pl.BlockSpec((_RT, D), lambda i: (i, 0)),
            pl.BlockSpec((K, _RT, D), lambda i: (0, i, 0)),
            pl.BlockSpec((_RT, D), lambda i: (i, 0)),
            pl.BlockSpec((D, D), lambda i: (0, 0)),
            pl.BlockSpec((1, D), lambda i: (0, 0)),
            pl.BlockSpec((D, D), lambda i: (0, 0)),
            pl.BlockSpec((1, D), lambda i: (0, 0)),
            pl.BlockSpec((1, D), lambda i: (0, 0)),
            pl.BlockSpec((1, D), lambda i: (0, 0)),
        ],
        out_specs=pl.BlockSpec((_RT, D), lambda i: (i, 0)),
        out_shape=jax.ShapeDtypeStruct((N, D), jnp.float32),
        interpret=_INTERPRET,
    )(a, bg, sc, w2, b2, w3, b3, s3, t3)


def _bn_fold(w, b):
    s = w / np.sqrt(1.0 + EPS)
    return s, b


def _dyn_edge_conv(x, batch, starts, p, c):
    nbr = _knn_pallas(x, batch, starts)
    s1, t1 = _bn_fold(p[c + 'bn1_w'], p[c + 'bn1_b'])
    s2, t2 = _bn_fold(p[c + 'bn2_w'], p[c + 'bn2_b'])
    s3, t3 = _bn_fold(p[c + 'bn3_w'], p[c + 'bn3_b'])
    ssc, tsc = _bn_fold(p[c + 'scbn_w'], p[c + 'scbn_b'])
    w1a = p[c + 'l1_w'][:D]
    w1b = p[c + 'l1_w'][D:]
    wa = w1a - w1b
    ba = p[c + 'l1_b'].reshape(1, H)
    w2f = s1[:, None] * p[c + 'l2_w']
    b2f = (t1 @ p[c + 'l2_w'] + p[c + 'l2_b']).reshape(1, H)
    w3f = s2[:, None] * p[c + 'l3_w']
    b3f = (t2 @ p[c + 'l3_w'] + p[c + 'l3_b']).reshape(1, H)
    wscf = p[c + 'sc_w'] * ssc[None, :]
    bscf = (p[c + 'sc_b'] * ssc + tsc).reshape(1, H)

    a, b, sc = _ab_sc(x, wa, ba, w1b, wscf, bscf)
    nbrT = jnp.zeros((K, _NP), jnp.int32).at[:, :N].set(nbr.T).reshape(-1)
    bg = _sc_gather(nbrT, b).reshape(K, _NP, D)
    return _econv(a, bg, sc, w2f, b2f, w3f, b3f,
                  s3.reshape(1, H), t3.reshape(1, H))


def _head_body(c1_ref, c2_ref, c3_ref, bcol_ref, gi_ref, w1g_ref, w1i_ref,
               b1_ref, w2_ref, b2_ref, wo_ref, bo_ref, out_ref):
    s = c1_ref[...] + c2_ref[...] + c3_ref[...]
    g_col = jax.lax.broadcasted_iota(jnp.int32, (G, 1), 0)
    ohT = (g_col == bcol_ref[...]).astype(jnp.float32)
    cnt = jnp.maximum(jnp.sum(ohT, axis=1, keepdims=True), 1.0)
    g = jax.lax.dot_general(ohT, s, (((1,), (0,)), ((), ())),
                            preferred_element_type=jnp.float32) / cnt
    g1 = jax.lax.dot_general(g, w1g_ref[...], (((1,), (0,)), ((), ())),
                             preferred_element_type=jnp.float32)
    g1 = g1 + jax.lax.dot_general(gi_ref[...], w1i_ref[...],
                                  (((1,), (0,)), ((), ())),
                                  preferred_element_type=jnp.float32)
    g1 = g1 + b1_ref[...]
    alpha = 1.6732632423543772
    scale = 1.0507009873554805
    g1 = scale * jnp.where(g1 > 0, g1,
                           alpha * (jnp.exp(jnp.minimum(g1, 0.0)) - 1.0))
    g2 = jax.lax.dot_general(g1, w2_ref[...], (((1,), (0,)), ((), ())),
                             preferred_element_type=jnp.float32) + b2_ref[...]
    g2 = scale * jnp.where(g2 > 0, g2,
                           alpha * (jnp.exp(jnp.minimum(g2, 0.0)) - 1.0))
    logits = jax.lax.dot_general(g2, wo_ref[...], (((1,), (0,)), ((), ())),
                                 preferred_element_type=jnp.float32) + bo_ref[...]
    lane = jax.lax.broadcasted_iota(jnp.int32, (G, 128), 1)
    ml = lane < C
    lm = jnp.where(ml, logits, -_INF)
    m = jnp.max(lm, axis=1, keepdims=True)
    ex = jnp.where(ml, jnp.exp(lm - m), 0.0)
    lse = jnp.log(jnp.sum(ex, axis=1, keepdims=True))
    out_ref[...] = logits - m - lse


def _head(c1, c2, c3, batch, graph_input, p):
    s0, t0 = _bn_fold(p['bn0_w'], p['bn0_b'])
    s0a, s0b = s0[:H], s0[H:]
    t0a, t0b = t0[:H], t0[H:]
    w1g = s0a[:, None] * p['d1_w'][:H]
    w1i = s0b[:, None] * p['d1_w'][H:]
    b1 = (t0a @ p['d1_w'][:H] + t0b @ p['d1_w'][H:] + p['d1_b']).reshape(1, H)
    w1i_p = jnp.zeros((128, H), jnp.float32).at[:GF].set(w1i)
    gi_p = jnp.zeros((G, 128), jnp.float32).at[:, :GF].set(graph_input)
    wo_p = jnp.zeros((H, 128), jnp.float32).at[:, :C].set(p['out_w'])
    bo_p = jnp.zeros((1, 128), jnp.float32).at[0, :C].set(p['out_b'])
    out = pl.pallas_call(
        _head_body,
        out_shape=jax.ShapeDtypeStruct((G, 128), jnp.float32),
        interpret=_INTERPRET,
    )(c1, c2, c3, batch.reshape(1, N), gi_p, w1g, w1i_p, b1,
      p['d2_w'], p['d2_b'].reshape(1, H), wo_p, bo_p)
    return out[:, :C]


_EP = 163840
_EPW = _EP // _NWK
_ECH = 128
_NCH = _EPW // _ECH
_ET = 2048
_AW = 144
_ISQH = float(1.0 / np.sqrt(float(H)))
E_REAL = 160000


def _edot_body(qd_ref, ks_ref, a_ref):
    a_ref[...] = jnp.sum(qd_ref[...] * ks_ref[...], axis=1, keepdims=True)


def _edot(qd, kvg):
    return pl.pallas_call(
        _edot_body,
        grid=(_EP // _ET,),
        in_specs=[pl.BlockSpec((_ET, H), lambda i: (i, 0)),
                  pl.BlockSpec((_ET, H), lambda i: (i, 0))],
        out_specs=pl.BlockSpec((_ET, 1), lambda i: (i, 0)),
        out_shape=jax.ShapeDtypeStruct((_EP, 1), jnp.float32),
        interpret=_INTERPRET,
    )(qd, kvg)


def _gmax_body(a_ref, out_ref):
    out_ref[...] = jnp.full((1, 128), jnp.max(a_ref[...]), jnp.float32)


def _gmax(a2d):
    return pl.pallas_call(
        _gmax_body,
        out_shape=jax.ShapeDtypeStruct((1, 128), jnp.float32),
        interpret=_INTERPRET,
    )(a2d)


def _st_body(a_ref, vs_ref, dm_ref, gm_ref, stn_ref, std_ref):
    i = pl.program_id(0)
    a = a_ref[...]
    ex = jnp.exp(a * _ISQH - gm_ref[0:1, 0:1])
    gid = i * _ET + jax.lax.broadcasted_iota(jnp.int32, (_ET, 1), 0)
    ex = jnp.where(gid < E_REAL, ex, 0.0)
    stn_ref[...] = ex * vs_ref[...]
    lane = jax.lax.broadcasted_iota(jnp.int32, (_ET, 128), 1)
    std_ref[...] = jnp.where(lane == dm_ref[...], ex, 0.0)


def _st_build(a, kvg, dmod, gm):
    return pl.pallas_call(
        _st_body,
        grid=(_EP // _ET,),
        in_specs=[pl.BlockSpec((_ET, 1), lambda i: (i, 0)),
                  pl.BlockSpec((_ET, H), lambda i: (i, 1)),
                  pl.BlockSpec((_ET, 1), lambda i: (i, 0)),
                  pl.BlockSpec((1, 128), lambda i: (0, 0))],
        out_specs=(pl.BlockSpec((_ET, H), lambda i: (i, 0)),
                   pl.BlockSpec((_ET, 128), lambda i: (i, 0))),
        out_shape=(jax.ShapeDtypeStruct((_EP, H), jnp.float32),
                   jax.ShapeDtypeStruct((_EP, 128), jnp.float32)),
        interpret=_INTERPRET,
    )(a, kvg, dmod, gm)


_DDIV = 128


def _sc_scatter_add(st, idx3, acc_rows):
    slab = acc_rows // 16

    def body(st_hbm, idx_hbm, z_hbm, acc_out, idxv, st0, st1, acc_sh, sem):
        cid = lax.axis_index("c")
        tid = lax.axis_index("s")
        wid = tid * 2 + cid
        base = wid * _EPW

        pltpu.sync_copy(z_hbm.at[pl.ds(0, slab)],
                        acc_sh.at[pl.ds(tid * slab, slab)])
        pltpu.sync_copy(idx_hbm.at[wid], idxv)
        plsc.subcore_barrier()

        sts = (st0, st1)

        def macro(m):
            cps = []
            for b in range(2):
                off = base + (m * 2 + b) * _ECH
                cps.append(pltpu.async_copy(st_hbm.at[pl.ds(off, _ECH)],
                                            sts[b], sem))
            cps[0].wait()
            pltpu.sync_copy(sts[0], acc_sh.at[idxv.at[m * 2]], add=True)
            cps[1].wait()
            pltpu.sync_copy(sts[1], acc_sh.at[idxv.at[m * 2 + 1]], add=True)

        pl.loop(0, _NCH // 2)(macro)
        plsc.subcore_barrier()
        pltpu.sync_copy(acc_sh.at[pl.ds(tid * slab, slab)],
                        acc_out.at[pl.ds(cid * acc_rows + tid * slab, slab)])

    z = jnp.zeros((max(_NP // 16, 128), 128), jnp.float32)
    mesh = plsc.VectorSubcoreMesh(core_axis_name="c", subcore_axis_name="s")
    f = pl.kernel(
        body,
        mesh=mesh,
        out_type=jax.ShapeDtypeStruct((2 * acc_rows, 128), jnp.float32),
        scratch_types=[
            pltpu.VMEM((_NCH, _ECH), jnp.int32),
            pltpu.VMEM((_ECH, 128), jnp.float32),
            pltpu.VMEM((_ECH, 128), jnp.float32),
            pltpu.VMEM_SHARED((acc_rows, 128), jnp.float32),
            pltpu.SemaphoreType.DMA,
        ],
    )
    return f(st, idx3, z)


def _attention_sc(q, kv, src, dst):
    srcp = jnp.zeros((_EP,), jnp.int32).at[:E_REAL].set(src)
    dstp = jnp.zeros((_EP,), jnp.int32).at[:E_REAL].set(dst)
    qd = _sc_gather(dstp, q)
    kvg = _sc_gather(srcp, kv)
    a = _edot(qd, kvg)
    gm = _gmax(a.reshape(_EP // 128, 128))
    stn, std = _st_build(a, kvg, (dstp % 128).reshape(_EP, 1), gm)
    accn = _sc_scatter_add(stn, dstp.reshape(_NWK, _NCH, _ECH), _NP)
    accd = _sc_scatter_add(std, (dstp // 128).reshape(_NWK, _NCH, _ECH),
                           _DDIV)
    num = accn.reshape(2, _NP, 128)
    den = (accd[:_DDIV] + accd[_DDIV:]).reshape(-1)[:_NP].reshape(_NP, 1)
    return num, den


def _gate_acc_body(acc_ref, den_ref, s_ref, wa_ref, wb_ref, c_ref):
    num = acc_ref[0] + acc_ref[1]
    den = den_ref[...]
    o = num / jnp.maximum(den, 1e-30)
    s = s_ref[...]
    z = jnp.sum(o * wa_ref[...] + s * wb_ref[...], axis=1, keepdims=True)
    bta = jax.nn.sigmoid(z)
    y = bta * s + (1.0 - bta) * o
    c_ref[...] = jnp.where(y > 0, y, jnp.exp(jnp.minimum(y, 0.0)) - 1.0)


def _gate_acc(num, den, skip, p):
    tb = p['tbeta_w']
    wa = (tb[0:128, 0] + tb[256:384, 0]).reshape(1, H)
    wb = (tb[128:256, 0] - tb[256:384, 0]).reshape(1, H)
    return pl.pallas_call(
        _gate_acc_body,
        grid=(N // _RT,),
        in_specs=[
            pl.BlockSpec((2, _RT, 128), lambda i: (0, i, 0)),
            pl.BlockSpec((_RT, 1), lambda i: (i, 0)),
            pl.BlockSpec((_RT, H), lambda i: (i, 0)),
            pl.BlockSpec((1, H), lambda i: (0, 0)),
            pl.BlockSpec((1, H), lambda i: (0, 0)),
        ],
        out_specs=pl.BlockSpec((_RT, H), lambda i: (i, 0)),
        out_shape=jax.ShapeDtypeStruct((N, H), jnp.float32),
        interpret=_INTERPRET,
    )(num, den, skip, wa, wb)


def kernel(x, edge_index, graph_input, batch, params):
    src, dst = edge_index[0], edge_index[1]
    h, starts = _graph_norm_starts(x, batch, params)
    q, kv, skip = _projections(h, params)
    num, den = _attention_sc(q, kv, src, dst)
    c1 = _gate_acc(num, den, skip, params)
    c2 = _dyn_edge_conv(c1, batch, starts, params, 'c2_')
    c3 = _dyn_edge_conv(c2, batch, starts, params, 'c3_')
    return _head(c1, c2, c3, batch, graph_input, params)


# speedup vs baseline: 1.1152x; 1.0309x over previous
import functools
from functools import partial

import jax
import jax.numpy as jnp
import numpy as np
from jax import lax
from jax.experimental import pallas as pl
from jax.experimental.pallas import tpu as pltpu
from jax.experimental.pallas import tpu_sc as plsc

N = 10000
D = 128
G = 64
GF = 16
H = 128
C = 10
K = 4
EPS = 1e-5

_INTERPRET = False

_NP = 10240
_RT = 1000


def _norm_body(x_ref, brow_ref, bcol_ref, ms_ref, w_ref, b_ref,
               h_ref, starts_ref):
    x = x_ref[...]
    brow = brow_ref[...]
    bcol = bcol_ref[...]
    g_row = jax.lax.broadcasted_iota(jnp.int32, (1, 128), 1)
    g_col = jax.lax.broadcasted_iota(jnp.int32, (128, 1), 0)
    oh = (brow == g_row).astype(jnp.float32)
    ohT = (g_col == bcol).astype(jnp.float32)
    cnt = jnp.sum(ohT, axis=1, keepdims=True)
    cnt = jnp.maximum(cnt, 1.0)
    sums = jax.lax.dot_general(ohT, x, (((1,), (0,)), ((), ())),
                               preferred_element_type=jnp.float32)
    mean = sums / cnt
    meanb = jax.lax.dot_general(oh, mean, (((1,), (0,)), ((), ())),
                                preferred_element_type=jnp.float32)
    xc = x - ms_ref[...] * meanb
    var = jax.lax.dot_general(ohT, xc * xc, (((1,), (0,)), ((), ())),
                              preferred_element_type=jnp.float32) / cnt
    std = jnp.sqrt(var + EPS)
    stdb = jax.lax.dot_general(oh, std, (((1,), (0,)), ((), ())),
                               preferred_element_type=jnp.float32)
    h_ref[...] = w_ref[...] * xc / stdb + b_ref[...]
    tri = (g_col < g_row).astype(jnp.float32)
    cnt_row = jnp.sum(oh, axis=0, keepdims=True)
    starts_f = jax.lax.dot_general(cnt_row, tri, (((1,), (0,)), ((), ())),
                                   preferred_element_type=jnp.float32)
    starts_ref[...] = starts_f.astype(jnp.int32)


def _graph_norm_starts(x, batch, p):
    brow = batch.reshape(N, 1)
    bcol = batch.reshape(1, N)
    ms = p['gn_ms'].reshape(1, D)
    w = p['gn_w'].reshape(1, D)
    b = p['gn_b'].reshape(1, D)
    h, starts = pl.pallas_call(
        _norm_body,
        out_shape=(jax.ShapeDtypeStruct((N, D), jnp.float32),
                   jax.ShapeDtypeStruct((1, 128), jnp.int32)),
        interpret=_INTERPRET,
    )(x, brow, bcol, ms, w, b)
    return h, starts[0, :G + 1]


def _proj_body(x_ref, w_ref, b_ref, q_ref, kv_ref, s_ref):
    y = jax.lax.dot_general(x_ref[...], w_ref[...], (((1,), (0,)), ((), ())),
                            preferred_element_type=jnp.float32) + b_ref[...]
    q_ref[...] = y[:, 0:128]
    kv_ref[...] = y[:, 128:384]
    s_ref[...] = y[:, 384:512]


def _projections(h, p):
    w4 = jnp.concatenate([p['tq_w'], p['tk_w'], p['tv_w'], p['ts_w']], axis=1)
    b4 = jnp.concatenate([p['tq_b'], p['tk_b'], p['tv_b'], p['ts_b']]
                         ).reshape(1, 512)
    outs = pl.pallas_call(
        _proj_body,
        grid=(N // _RT,),
        in_specs=[
            pl.BlockSpec((_RT, D), lambda i: (i, 0)),
            pl.BlockSpec((D, 512), lambda i: (0, 0)),
            pl.BlockSpec((1, 512), lambda i: (0, 0)),
        ],
        out_specs=(pl.BlockSpec((_RT, H), lambda i: (i, 0)),
                   pl.BlockSpec((_RT, 256), lambda i: (i, 0)),
                   pl.BlockSpec((_RT, H), lambda i: (i, 0))),
        out_shape=(jax.ShapeDtypeStruct((N, H), jnp.float32),
                   jax.ShapeDtypeStruct((N, 256), jnp.float32),
                   jax.ShapeDtypeStruct((N, H), jnp.float32)),
        interpret=_INTERPRET,
    )(h, w4, b4)
    return outs


def _gate_body(o_ref, s_ref, wa_ref, wb_ref, c_ref):
    o = o_ref[...]
    s = s_ref[...]
    z = jnp.sum(o * wa_ref[...] + s * wb_ref[...], axis=1, keepdims=True)
    bta = jax.nn.sigmoid(z)
    y = bta * s + (1.0 - bta) * o
    c_ref[...] = jnp.where(y > 0, y, jnp.exp(jnp.minimum(y, 0.0)) - 1.0)


def _gate(out, skip, p):
    tb = p['tbeta_w']
    wa = (tb[0:128, 0] + tb[256:384, 0]).reshape(1, H)
    wb = (tb[128:256, 0] - tb[256:384, 0]).reshape(1, H)
    return pl.pallas_call(
        _gate_body,
        grid=(N // _RT,),
        in_specs=[
            pl.BlockSpec((_RT, H), lambda i: (i, 0)),
            pl.BlockSpec((_RT, H), lambda i: (i, 0)),
            pl.BlockSpec((1, H), lambda i: (0, 0)),
            pl.BlockSpec((1, H), lambda i: (0, 0)),
        ],
        out_specs=pl.BlockSpec((_RT, H), lambda i: (i, 0)),
        out_shape=jax.ShapeDtypeStruct((N, H), jnp.float32),
        interpret=_INTERPRET,
    )(out, skip, wa, wb)


_R = 256
_CW = 512
_BIG = 1e30
_INF = np.float32(np.inf)


def _knn_body(batch_sm, starts_sm, x_ref, xt_ref, bcol_ref, sqcol_ref,
              brow_ref, out_ref):
    i = pl.program_id(0)
    xi = x_ref[...]
    bi = brow_ref[...]
    r0 = i * _R
    b_lo = jnp.clip(batch_sm[jnp.minimum(r0, N - 1)], 0, G - 1)
    b_hi = jnp.clip(batch_sm[jnp.minimum(r0 + _R - 1, N - 1)], 0, G - 1)
    jstart = starts_sm[b_lo]
    jend = starts_sm[b_hi + 1]
    j0 = jstart // _CW
    j1 = (jend + _CW - 1) // _CW

    row_ids = r0 + jax.lax.broadcasted_iota(jnp.int32, (_R, 1), 0)
    lane128 = jax.lax.broadcasted_iota(jnp.int32, (_R, 128), 1)
    run_v = jnp.where(lane128 < K, jnp.float32(_BIG), _INF)
    run_i = jnp.where(lane128 < K, lane128, 0)

    W = 128 + _CW
    lane = jax.lax.broadcasted_iota(jnp.int32, (_R, W), 1)

    def body(j, carry):
        run_v, run_i = carry
        c0 = pl.multiple_of(j * _CW, _CW)
        xj = xt_ref[:, pl.ds(c0, _CW)]
        prod = jax.lax.dot_general(
            xi, xj, (((1,), (0,)), ((), ())),
            preferred_element_type=jnp.float32)
        sqj = sqcol_ref[:, pl.ds(c0, _CW)]
        bj = bcol_ref[:, pl.ds(c0, _CW)]
        col_ids = c0 + jax.lax.broadcasted_iota(jnp.int32, (1, _CW), 1)
        d2 = sqj - 2.0 * prod
        d2 = jnp.where(bi != bj, jnp.float32(_BIG), d2)
        d2 = jnp.where(row_ids == col_ids, jnp.float32(_BIG), d2)
        cand_v = jnp.concatenate([run_v, d2], axis=1)
        cand_i = jnp.concatenate(
            [run_i, jnp.broadcast_to(col_ids, (_R, _CW))], axis=1)
        new_v, new_i = [], []
        for _ in range(K):
            m = jnp.min(cand_v, axis=1, keepdims=True)
            posm = jnp.where(cand_v == m, lane, W)
            pos = jnp.min(posm, axis=1, keepdims=True)
            sel = lane == pos
            idxk = jnp.max(jnp.where(sel, cand_i, -1), axis=1, keepdims=True)
            new_v.append(m)
            new_i.append(idxk)
            cand_v = jnp.where(sel, _INF, cand_v)
        pad_v = jnp.full((_R, 128 - K), _INF, jnp.float32)
        pad_i = jnp.zeros((_R, 128 - K), jnp.int32)
        return (jnp.concatenate(new_v + [pad_v], axis=1),
                jnp.concatenate(new_i + [pad_i], axis=1))

    run_v, run_i = jax.lax.fori_loop(j0, j1, body, (run_v, run_i))
    out_ref[...] = run_i


def _knn_pallas(x, batch, starts):
    xp = jnp.zeros((_NP, D), jnp.float32).at[:N].set(x)
    xt = jnp.zeros((D, _NP), jnp.float32).at[:, :N].set(x.T)
    bcol = jnp.full((1, _NP), -1, jnp.int32).at[0, :N].set(batch)
    brow = jnp.full((_NP, 1), G, jnp.int32).at[:N, 0].set(batch)
    sqcol = jnp.zeros((1, _NP), jnp.float32).at[0, :N].set(
        jnp.sum(x * x, axis=1))
    grid_spec = pltpu.PrefetchScalarGridSpec(
        num_scalar_prefetch=2,
        grid=(_NP // _R,),
        in_specs=[
            pl.BlockSpec((_R, D), lambda i, *_: (i, 0)),
            pl.BlockSpec((D, _NP), lambda i, *_: (0, 0)),
            pl.BlockSpec((1, _NP), lambda i, *_: (0, 0)),
            pl.BlockSpec((1, _NP), lambda i, *_: (0, 0)),
            pl.BlockSpec((_R, 1), lambda i, *_: (i, 0)),
        ],
        out_specs=pl.BlockSpec((_R, 128), lambda i, *_: (i, 0)),
    )
    out = pl.pallas_call(
        _knn_body,
        grid_spec=grid_spec,
        out_shape=jax.ShapeDtypeStruct((_NP, 128), jnp.int32),
        interpret=_INTERPRET,
    )(batch, starts, xp, xt, bcol, sqcol, brow)
    return out[:N, :K]


def _ab_body(x_ref, wa_ref, ba_ref, wb_ref, wsc_ref, bsc_ref,
             a_ref, b_ref, sc_ref):
    x = x_ref[...]
    a_ref[...] = jax.lax.dot_general(
        x, wa_ref[...], (((1,), (0,)), ((), ())),
        preferred_element_type=jnp.float32) + ba_ref[...]
    b_ref[...] = jax.lax.dot_general(
        x, wb_ref[...], (((1,), (0,)), ((), ())),
        preferred_element_type=jnp.float32)
    sc_ref[...] = jax.lax.dot_general(
        x, wsc_ref[...], (((1,), (0,)), ((), ())),
        preferred_element_type=jnp.float32) + bsc_ref[...]


def _ab_sc(x, wa, ba, wb, wsc, bsc):
    return pl.pallas_call(
        _ab_body,
        grid=(N // _RT,),
        in_specs=[pl.BlockSpec((_RT, D), lambda i: (i, 0))] +
                 [pl.BlockSpec((D, D), lambda i: (0, 0)),
                  pl.BlockSpec((1, D), lambda i: (0, 0)),
                  pl.BlockSpec((D, D), lambda i: (0, 0)),
                  pl.BlockSpec((D, D), lambda i: (0, 0)),
                  pl.BlockSpec((1, D), lambda i: (0, 0))],
        out_specs=tuple(pl.BlockSpec((_RT, D), lambda i: (i, 0))
                        for _ in range(3)),
        out_shape=tuple(jax.ShapeDtypeStruct((N, D), jnp.float32)
                        for _ in range(3)),
        interpret=_INTERPRET,
    )(x, wa, ba, wb, wsc, bsc)


_NWK = 32
_RPW = _NP // _NWK


_GCH = 128
_C0_SHARE_PCT = 75


def _sc_gather(idx, table):
    M = idx.shape[0]
    Wd = table.shape[1]
    rpw = M // _NWK
    nch = rpw // _GCH
    nb = 4 if (nch % 4 == 0 and Wd <= 128) else 2
    w0 = max(nb, ((2 * nch * _C0_SHARE_PCT // 100) // nb) * nb)

    def body(idx_hbm, table_hbm, out_hbm, idx_v, *rows_sem):
        rows, sem = rows_sem[:nb], rows_sem[nb]
        sid = lax.axis_index("s")
        cid = lax.axis_index("c")
        pair_first = sid * 2 * nch
        my_cnt = jnp.where(cid == 0, w0, 2 * nch - w0)
        my_loc = jnp.where(cid == 0, 0, w0)
        pltpu.sync_copy(idx_hbm.at[pl.ds(pair_first * _GCH, 2 * rpw)], idx_v)

        def macro(m):
            cps = []
            for b in range(nb):
                loc = (my_loc + m * nb + b) * _GCH
                cps.append(pltpu.async_copy(
                    table_hbm.at[idx_v.at[pl.ds(loc, _GCH)]],
                    rows[b], sem))
            for cp in cps:
                cp.wait()
            for b in range(nb):
                loc = (my_loc + m * nb + b) * _GCH
                pltpu.sync_copy(rows[b],
                                out_hbm.at[pl.ds(pair_first * _GCH + loc,
                                                 _GCH)])

        pl.loop(0, my_cnt // nb)(macro)

    mesh = plsc.VectorSubcoreMesh(core_axis_name="c", subcore_axis_name="s")
    f = pl.kernel(
        body,
        mesh=mesh,
        out_type=jax.ShapeDtypeStruct((M, Wd), jnp.float32),
        scratch_types=[pltpu.VMEM((2 * rpw,), jnp.int32)] +
                      [pltpu.VMEM((_GCH, Wd), jnp.float32)
                       for _ in range(nb)] +
                      [pltpu.SemaphoreType.DMA],
    )
    return f(idx, table)


def _econv_body(a_ref, bg_ref, sc_ref, w2_ref, b2_ref, w3_ref, b3_ref,
                s3_ref, t3_ref, out_ref):
    a = a_ref[...]
    acc = None
    for k in range(K):
        h = jnp.maximum(a + bg_ref[k], 0.0)
        h = jnp.maximum(jax.lax.dot_general(
            h, w2_ref[...], (((1,), (0,)), ((), ())),
            preferred_element_type=jnp.float32) + b2_ref[...], 0.0)
        h = jnp.maximum(jax.lax.dot_general(
            h, w3_ref[...], (((1,), (0,)), ((), ())),
            preferred_element_type=jnp.float32) + b3_ref[...], 0.0)
        h = h * s3_ref[...] + t3_ref[...]
        acc = h if acc is None else jnp.maximum(acc, h)
    out_ref[...] = acc + sc_ref[...]


def _econv(a, bg, sc, w2, b2, w3, b3, s3, t3):
    return pl.pallas_call(
        _econv_body,
        grid=(N // _RT,),
        in_specs=[
            pl.BlockSpec((_RT, D), lambda i: (i, 0)),
            pl.BlockSpec((K, _RT, D), lambda i: (0, i, 0)),
            pl.BlockSpec((_RT, D), lambda i: (i, 0)),
            pl.BlockSpec((D, D), lambda i: (0, 0)),
            pl.BlockSpec((1, D), lambda i: (0, 0)),
            pl.BlockSpec((D, D), lambda i: (0, 0)),
            pl.BlockSpec((1, D), lambda i: (0, 0)),
            pl.BlockSpec((1, D), lambda i: (0, 0)),
            pl.BlockSpec((1, D), lambda i: (0, 0)),
        ],
        out_specs=pl.BlockSpec((_RT, D), lambda i: (i, 0)),
        out_shape=jax.ShapeDtypeStruct((N, D), jnp.float32),
        interpret=_INTERPRET,
    )(a, bg, sc, w2, b2, w3, b3, s3, t3)


def _bn_fold(w, b):
    s = w / np.sqrt(1.0 + EPS)
    return s, b


def _dyn_edge_conv(x, batch, starts, p, c):
    nbr = _knn_pallas(x, batch, starts)
    s1, t1 = _bn_fold(p[c + 'bn1_w'], p[c + 'bn1_b'])
    s2, t2 = _bn_fold(p[c + 'bn2_w'], p[c + 'bn2_b'])
    s3, t3 = _bn_fold(p[c + 'bn3_w'], p[c + 'bn3_b'])
    ssc, tsc = _bn_fold(p[c + 'scbn_w'], p[c + 'scbn_b'])
    w1a = p[c + 'l1_w'][:D]
    w1b = p[c + 'l1_w'][D:]
    wa = w1a - w1b
    ba = p[c + 'l1_b'].reshape(1, H)
    w2f = s1[:, None] * p[c + 'l2_w']
    b2f = (t1 @ p[c + 'l2_w'] + p[c + 'l2_b']).reshape(1, H)
    w3f = s2[:, None] * p[c + 'l3_w']
    b3f = (t2 @ p[c + 'l3_w'] + p[c + 'l3_b']).reshape(1, H)
    wscf = p[c + 'sc_w'] * ssc[None, :]
    bscf = (p[c + 'sc_b'] * ssc + tsc).reshape(1, H)

    a, b, sc = _ab_sc(x, wa, ba, w1b, wscf, bscf)
    nbrT = jnp.zeros((K, _NP), jnp.int32).at[:, :N].set(nbr.T).reshape(-1)
    bg = _sc_gather(nbrT, b).reshape(K, _NP, D)
    return _econv(a, bg, sc, w2f, b2f, w3f, b3f,
                  s3.reshape(1, H), t3.reshape(1, H))


def _head_body(c1_ref, c2_ref, c3_ref, bcol_ref, gi_ref, w1g_ref, w1i_ref,
               b1_ref, w2_ref, b2_ref, wo_ref, bo_ref, out_ref):
    s = c1_ref[...] + c2_ref[...] + c3_ref[...]
    g_col = jax.lax.broadcasted_iota(jnp.int32, (G, 1), 0)
    ohT = (g_col == bcol_ref[...]).astype(jnp.float32)
    cnt = jnp.maximum(jnp.sum(ohT, axis=1, keepdims=True), 1.0)
    g = jax.lax.dot_general(ohT, s, (((1,), (0,)), ((), ())),
                            preferred_element_type=jnp.float32) / cnt
    g1 = jax.lax.dot_general(g, w1g_ref[...], (((1,), (0,)), ((), ())),
                             preferred_element_type=jnp.float32)
    g1 = g1 + jax.lax.dot_general(gi_ref[...], w1i_ref[...],
                                  (((1,), (0,)), ((), ())),
                                  preferred_element_type=jnp.float32)
    g1 = g1 + b1_ref[...]
    alpha = 1.6732632423543772
    scale = 1.0507009873554805
    g1 = scale * jnp.where(g1 > 0, g1,
                           alpha * (jnp.exp(jnp.minimum(g1, 0.0)) - 1.0))
    g2 = jax.lax.dot_general(g1, w2_ref[...], (((1,), (0,)), ((), ())),
                             preferred_element_type=jnp.float32) + b2_ref[...]
    g2 = scale * jnp.where(g2 > 0, g2,
                           alpha * (jnp.exp(jnp.minimum(g2, 0.0)) - 1.0))
    logits = jax.lax.dot_general(g2, wo_ref[...], (((1,), (0,)), ((), ())),
                                 preferred_element_type=jnp.float32) + bo_ref[...]
    lane = jax.lax.broadcasted_iota(jnp.int32, (G, 128), 1)
    ml = lane < C
    lm = jnp.where(ml, logits, -_INF)
    m = jnp.max(lm, axis=1, keepdims=True)
    ex = jnp.where(ml, jnp.exp(lm - m), 0.0)
    lse = jnp.log(jnp.sum(ex, axis=1, keepdims=True))
    out_ref[...] = logits - m - lse


def _head(c1, c2, c3, batch, graph_input, p):
    s0, t0 = _bn_fold(p['bn0_w'], p['bn0_b'])
    s0a, s0b = s0[:H], s0[H:]
    t0a, t0b = t0[:H], t0[H:]
    w1g = s0a[:, None] * p['d1_w'][:H]
    w1i = s0b[:, None] * p['d1_w'][H:]
    b1 = (t0a @ p['d1_w'][:H] + t0b @ p['d1_w'][H:] + p['d1_b']).reshape(1, H)
    w1i_p = jnp.zeros((128, H), jnp.float32).at[:GF].set(w1i)
    gi_p = jnp.zeros((G, 128), jnp.float32).at[:, :GF].set(graph_input)
    wo_p = jnp.zeros((H, 128), jnp.float32).at[:, :C].set(p['out_w'])
    bo_p = jnp.zeros((1, 128), jnp.float32).at[0, :C].set(p['out_b'])
    out = pl.pallas_call(
        _head_body,
        out_shape=jax.ShapeDtypeStruct((G, 128), jnp.float32),
        interpret=_INTERPRET,
    )(c1, c2, c3, batch.reshape(1, N), gi_p, w1g, w1i_p, b1,
      p['d2_w'], p['d2_b'].reshape(1, H), wo_p, bo_p)
    return out[:, :C]


_EP = 163840
_EPW = _EP // _NWK
_ECH = 128
_NCH = _EPW // _ECH
_ET = 2048
_AW = 144
_ISQH = float(1.0 / np.sqrt(float(H)))
E_REAL = 160000


def _edot_body(qd_ref, ks_ref, a_ref):
    a_ref[...] = jnp.sum(qd_ref[...] * ks_ref[...], axis=1, keepdims=True)


def _edot(qd, kvg):
    return pl.pallas_call(
        _edot_body,
        grid=(_EP // _ET,),
        in_specs=[pl.BlockSpec((_ET, H), lambda i: (i, 0)),
                  pl.BlockSpec((_ET, H), lambda i: (i, 0))],
        out_specs=pl.BlockSpec((_ET, 1), lambda i: (i, 0)),
        out_shape=jax.ShapeDtypeStruct((_EP, 1), jnp.float32),
        interpret=_INTERPRET,
    )(qd, kvg)


def _gmax_body(a_ref, out_ref):
    out_ref[...] = jnp.full((1, 128), jnp.max(a_ref[...]), jnp.float32)


def _gmax(a2d):
    return pl.pallas_call(
        _gmax_body,
        out_shape=jax.ShapeDtypeStruct((1, 128), jnp.float32),
        interpret=_INTERPRET,
    )(a2d)


def _st_body(a_ref, vs_ref, dm_ref, gm_ref, stn_ref, std_ref):
    i = pl.program_id(0)
    a = a_ref[...]
    ex = jnp.exp(a * _ISQH - gm_ref[0:1, 0:1])
    gid = i * _ET + jax.lax.broadcasted_iota(jnp.int32, (_ET, 1), 0)
    ex = jnp.where(gid < E_REAL, ex, 0.0)
    stn_ref[...] = ex * vs_ref[...]
    lane = jax.lax.broadcasted_iota(jnp.int32, (_ET, 128), 1)
    std_ref[...] = jnp.where(lane == dm_ref[...], ex, 0.0)


def _st_build(a, kvg, dmod, gm):
    return pl.pallas_call(
        _st_body,
        grid=(_EP // _ET,),
        in_specs=[pl.BlockSpec((_ET, 1), lambda i: (i, 0)),
                  pl.BlockSpec((_ET, H), lambda i: (i, 1)),
                  pl.BlockSpec((_ET, 1), lambda i: (i, 0)),
                  pl.BlockSpec((1, 128), lambda i: (0, 0))],
        out_specs=(pl.BlockSpec((_ET, H), lambda i: (i, 0)),
                   pl.BlockSpec((_ET, 128), lambda i: (i, 0))),
        out_shape=(jax.ShapeDtypeStruct((_EP, H), jnp.float32),
                   jax.ShapeDtypeStruct((_EP, 128), jnp.float32)),
        interpret=_INTERPRET,
    )(a, kvg, dmod, gm)


_DDIV = 128


def _sc_scatter_add(st, idx3, acc_rows):
    slab = acc_rows // 16

    def body(st_hbm, idx_hbm, z_hbm, acc_out, idxv, st0, st1, acc_sh, sem):
        cid = lax.axis_index("c")
        tid = lax.axis_index("s")
        wid = tid * 2 + cid
        base = wid * _EPW

        pltpu.sync_copy(z_hbm.at[pl.ds(0, slab)],
                        acc_sh.at[pl.ds(tid * slab, slab)])
        pltpu.sync_copy(idx_hbm.at[wid], idxv)
        plsc.subcore_barrier()

        sts = (st0, st1)

        def macro(m):
            cps = []
            for b in range(2):
                off = base + (m * 2 + b) * _ECH
                cps.append(pltpu.async_copy(st_hbm.at[pl.ds(off, _ECH)],
                                            sts[b], sem))
            cps[0].wait()
            pltpu.sync_copy(sts[0], acc_sh.at[idxv.at[m * 2]], add=True)
            cps[1].wait()
            pltpu.sync_copy(sts[1], acc_sh.at[idxv.at[m * 2 + 1]], add=True)

        pl.loop(0, _NCH // 2)(macro)
        plsc.subcore_barrier()
        pltpu.sync_copy(acc_sh.at[pl.ds(tid * slab, slab)],
                        acc_out.at[pl.ds(cid * acc_rows + tid * slab, slab)])

    z = jnp.zeros((max(_NP // 16, 128), 128), jnp.float32)
    mesh = plsc.VectorSubcoreMesh(core_axis_name="c", subcore_axis_name="s")
    f = pl.kernel(
        body,
        mesh=mesh,
        out_type=jax.ShapeDtypeStruct((2 * acc_rows, 128), jnp.float32),
        scratch_types=[
            pltpu.VMEM((_NCH, _ECH), jnp.int32),
            pltpu.VMEM((_ECH, 128), jnp.float32),
            pltpu.VMEM((_ECH, 128), jnp.float32),
            pltpu.VMEM_SHARED((acc_rows, 128), jnp.float32),
            pltpu.SemaphoreType.DMA,
        ],
    )
    return f(st, idx3, z)


def _attention_sc(q, kv, src, dst):
    srcp = jnp.zeros((_EP,), jnp.int32).at[:E_REAL].set(src)
    dstp = jnp.zeros((_EP,), jnp.int32).at[:E_REAL].set(dst)
    qd = _sc_gather(dstp, q)
    kvg = _sc_gather(srcp, kv)
    a = _edot(qd, kvg)
    gm = _gmax(a.reshape(_EP // 128, 128))
    stn, std = _st_build(a, kvg, (dstp % 128).reshape(_EP, 1), gm)
    accn = _sc_scatter_add(stn, dstp.reshape(_NWK, _NCH, _ECH), _NP)
    accd = _sc_scatter_add(std, (dstp // 128).reshape(_NWK, _NCH, _ECH),
                           _DDIV)
    num = accn.reshape(2, _NP, 128)
    den = (accd[:_DDIV] + accd[_DDIV:]).reshape(-1)[:_NP].reshape(_NP, 1)
    return num, den


def _gate_acc_body(acc_ref, den_ref, s_ref, wa_ref, wb_ref, c_ref):
    num = acc_ref[0] + acc_ref[1]
    den = den_ref[...]
    o = num / jnp.maximum(den, 1e-30)
    s = s_ref[...]
    z = jnp.sum(o * wa_ref[...] + s * wb_ref[...], axis=1, keepdims=True)
    bta = jax.nn.sigmoid(z)
    y = bta * s + (1.0 - bta) * o
    c_ref[...] = jnp.where(y > 0, y, jnp.exp(jnp.minimum(y, 0.0)) - 1.0)


def _gate_acc(num, den, skip, p):
    tb = p['tbeta_w']
    wa = (tb[0:128, 0] + tb[256:384, 0]).reshape(1, H)
    wb = (tb[128:256, 0] - tb[256:384, 0]).reshape(1, H)
    return pl.pallas_call(
        _gate_acc_body,
        grid=(N // _RT,),
        in_specs=[
            pl.BlockSpec((2, _RT, 128), lambda i: (0, i, 0)),
            pl.BlockSpec((_RT, 1), lambda i: (i, 0)),
            pl.BlockSpec((_RT, H), lambda i: (i, 0)),
            pl.BlockSpec((1, H), lambda i: (0, 0)),
            pl.BlockSpec((1, H), lambda i: (0, 0)),
        ],
        out_specs=pl.BlockSpec((_RT, H), lambda i: (i, 0)),
        out_shape=jax.ShapeDtypeStruct((N, H), jnp.float32),
        interpret=_INTERPRET,
    )(num, den, skip, wa, wb)


def kernel(x, edge_index, graph_input, batch, params):
    src, dst = edge_index[0], edge_index[1]
    h, starts = _graph_norm_starts(x, batch, params)
    q, kv, skip = _projections(h, params)
    num, den = _attention_sc(q, kv, src, dst)
    c1 = _gate_acc(num, den, skip, params)
    c2 = _dyn_edge_conv(c1, batch, starts, params, 'c2_')
    c3 = _dyn_edge_conv(c2, batch, starts, params, 'c3_')
    return _head(c1, c2, c3, batch, graph_input, params)


# speedup vs baseline: 1.1154x; 1.0002x over previous
import jax
import jax.numpy as jnp
import numpy as np
from jax import lax
from jax.experimental import pallas as pl
from jax.experimental.pallas import tpu as pltpu
from jax.experimental.pallas import tpu_sc as plsc

N = 10000
D = 128
G = 64
GF = 16
H = 128
C = 10
K = 4
EPS = 1e-5

_INTERPRET = False

_NP = 10240
_RT = 1000


def _norm_body(x_ref, brow_ref, bcol_ref, ms_ref, w_ref, b_ref,
               h_ref, starts_ref):
    x = x_ref[...]
    brow = brow_ref[...]
    bcol = bcol_ref[...]
    g_row = jax.lax.broadcasted_iota(jnp.int32, (1, 128), 1)
    g_col = jax.lax.broadcasted_iota(jnp.int32, (128, 1), 0)
    oh = (brow == g_row).astype(jnp.float32)
    ohT = (g_col == bcol).astype(jnp.float32)
    cnt = jnp.sum(ohT, axis=1, keepdims=True)
    cnt = jnp.maximum(cnt, 1.0)
    sums = jax.lax.dot_general(ohT, x, (((1,), (0,)), ((), ())),
                               preferred_element_type=jnp.float32)
    mean = sums / cnt
    meanb = jax.lax.dot_general(oh, mean, (((1,), (0,)), ((), ())),
                                preferred_element_type=jnp.float32)
    xc = x - ms_ref[...] * meanb
    var = jax.lax.dot_general(ohT, xc * xc, (((1,), (0,)), ((), ())),
                              preferred_element_type=jnp.float32) / cnt
    std = jnp.sqrt(var + EPS)
    stdb = jax.lax.dot_general(oh, std, (((1,), (0,)), ((), ())),
                               preferred_element_type=jnp.float32)
    h_ref[...] = w_ref[...] * xc / stdb + b_ref[...]
    tri = (g_col < g_row).astype(jnp.float32)
    cnt_row = jnp.sum(oh, axis=0, keepdims=True)
    starts_f = jax.lax.dot_general(cnt_row, tri, (((1,), (0,)), ((), ())),
                                   preferred_element_type=jnp.float32)
    starts_ref[...] = starts_f.astype(jnp.int32)


def _graph_norm_starts(x, batch, p):
    brow = batch.reshape(N, 1)
    bcol = batch.reshape(1, N)
    ms = p['gn_ms'].reshape(1, D)
    w = p['gn_w'].reshape(1, D)
    b = p['gn_b'].reshape(1, D)
    h, starts = pl.pallas_call(
        _norm_body,
        out_shape=(jax.ShapeDtypeStruct((N, D), jnp.float32),
                   jax.ShapeDtypeStruct((1, 128), jnp.int32)),
        interpret=_INTERPRET,
    )(x, brow, bcol, ms, w, b)
    return h, starts[0, :G + 1]


def _proj_body(x_ref, w_ref, b_ref, q_ref, kv_ref, s_ref):
    y = jax.lax.dot_general(x_ref[...], w_ref[...], (((1,), (0,)), ((), ())),
                            preferred_element_type=jnp.float32) + b_ref[...]
    q_ref[...] = y[:, 0:128]
    kv_ref[...] = y[:, 128:384]
    s_ref[...] = y[:, 384:512]


def _projections(h, p):
    w4 = jnp.concatenate([p['tq_w'], p['tk_w'], p['tv_w'], p['ts_w']], axis=1)
    b4 = jnp.concatenate([p['tq_b'], p['tk_b'], p['tv_b'], p['ts_b']]
                         ).reshape(1, 512)
    outs = pl.pallas_call(
        _proj_body,
        grid=(N // _RT,),
        in_specs=[
            pl.BlockSpec((_RT, D), lambda i: (i, 0)),
            pl.BlockSpec((D, 512), lambda i: (0, 0)),
            pl.BlockSpec((1, 512), lambda i: (0, 0)),
        ],
        out_specs=(pl.BlockSpec((_RT, H), lambda i: (i, 0)),
                   pl.BlockSpec((_RT, 256), lambda i: (i, 0)),
                   pl.BlockSpec((_RT, H), lambda i: (i, 0))),
        out_shape=(jax.ShapeDtypeStruct((N, H), jnp.float32),
                   jax.ShapeDtypeStruct((N, 256), jnp.float32),
                   jax.ShapeDtypeStruct((N, H), jnp.float32)),
        interpret=_INTERPRET,
    )(h, w4, b4)
    return outs


_R = 256
_CW = 512
_BIG = 1e30
_INF = np.float32(np.inf)


def _knn_body(batch_sm, starts_sm, x_ref, xt_ref, bcol_ref, sqcol_ref,
              brow_ref, out_ref):
    i = pl.program_id(0)
    xi = x_ref[...]
    bi = brow_ref[...]
    r0 = i * _R
    b_lo = jnp.clip(batch_sm[jnp.minimum(r0, N - 1)], 0, G - 1)
    b_hi = jnp.clip(batch_sm[jnp.minimum(r0 + _R - 1, N - 1)], 0, G - 1)
    jstart = starts_sm[b_lo]
    jend = starts_sm[b_hi + 1]
    j0 = jstart // _CW
    j1 = (jend + _CW - 1) // _CW

    row_ids = r0 + jax.lax.broadcasted_iota(jnp.int32, (_R, 1), 0)
    lane128 = jax.lax.broadcasted_iota(jnp.int32, (_R, 128), 1)
    run_v = jnp.where(lane128 < K, jnp.float32(_BIG), _INF)
    run_i = jnp.where(lane128 < K, lane128, 0)

    W = 128 + _CW
    lane = jax.lax.broadcasted_iota(jnp.int32, (_R, W), 1)

    def body(j, carry):
        run_v, run_i = carry
        c0 = pl.multiple_of(j * _CW, _CW)
        xj = xt_ref[:, pl.ds(c0, _CW)]
        prod = jax.lax.dot_general(
            xi, xj, (((1,), (0,)), ((), ())),
            preferred_element_type=jnp.float32)
        sqj = sqcol_ref[:, pl.ds(c0, _CW)]
        bj = bcol_ref[:, pl.ds(c0, _CW)]
        col_ids = c0 + jax.lax.broadcasted_iota(jnp.int32, (1, _CW), 1)
        d2 = sqj - 2.0 * prod
        d2 = jnp.where(bi != bj, jnp.float32(_BIG), d2)
        d2 = jnp.where(row_ids == col_ids, jnp.float32(_BIG), d2)
        cand_v = jnp.concatenate([run_v, d2], axis=1)
        cand_i = jnp.concatenate(
            [run_i, jnp.broadcast_to(col_ids, (_R, _CW))], axis=1)
        new_v, new_i = [], []
        for _ in range(K):
            m = jnp.min(cand_v, axis=1, keepdims=True)
            posm = jnp.where(cand_v == m, lane, W)
            pos = jnp.min(posm, axis=1, keepdims=True)
            sel = lane == pos
            idxk = jnp.max(jnp.where(sel, cand_i, -1), axis=1, keepdims=True)
            new_v.append(m)
            new_i.append(idxk)
            cand_v = jnp.where(sel, _INF, cand_v)
        pad_v = jnp.full((_R, 128 - K), _INF, jnp.float32)
        pad_i = jnp.zeros((_R, 128 - K), jnp.int32)
        return (jnp.concatenate(new_v + [pad_v], axis=1),
                jnp.concatenate(new_i + [pad_i], axis=1))

    run_v, run_i = jax.lax.fori_loop(j0, j1, body, (run_v, run_i))
    out_ref[...] = run_i


def _knn_pallas(x, batch, starts):
    xp = jnp.zeros((_NP, D), jnp.float32).at[:N].set(x)
    xt = jnp.zeros((D, _NP), jnp.float32).at[:, :N].set(x.T)
    bcol = jnp.full((1, _NP), -1, jnp.int32).at[0, :N].set(batch)
    brow = jnp.full((_NP, 1), G, jnp.int32).at[:N, 0].set(batch)
    sqcol = jnp.zeros((1, _NP), jnp.float32).at[0, :N].set(
        jnp.sum(x * x, axis=1))
    grid_spec = pltpu.PrefetchScalarGridSpec(
        num_scalar_prefetch=2,
        grid=(_NP // _R,),
        in_specs=[
            pl.BlockSpec((_R, D), lambda i, *_: (i, 0)),
            pl.BlockSpec((D, _NP), lambda i, *_: (0, 0)),
            pl.BlockSpec((1, _NP), lambda i, *_: (0, 0)),
            pl.BlockSpec((1, _NP), lambda i, *_: (0, 0)),
            pl.BlockSpec((_R, 1), lambda i, *_: (i, 0)),
        ],
        out_specs=pl.BlockSpec((_R, 128), lambda i, *_: (i, 0)),
    )
    out = pl.pallas_call(
        _knn_body,
        grid_spec=grid_spec,
        out_shape=jax.ShapeDtypeStruct((_NP, 128), jnp.int32),
        interpret=_INTERPRET,
    )(batch, starts, xp, xt, bcol, sqcol, brow)
    return out[:N, :K]


def _ab_body(x_ref, wa_ref, ba_ref, wb_ref, wsc_ref, bsc_ref,
             a_ref, b_ref, sc_ref):
    x = x_ref[...]
    a_ref[...] = jax.lax.dot_general(
        x, wa_ref[...], (((1,), (0,)), ((), ())),
        preferred_element_type=jnp.float32) + ba_ref[...]
    b_ref[...] = jax.lax.dot_general(
        x, wb_ref[...], (((1,), (0,)), ((), ())),
        preferred_element_type=jnp.float32)
    sc_ref[...] = jax.lax.dot_general(
        x, wsc_ref[...], (((1,), (0,)), ((), ())),
        preferred_element_type=jnp.float32) + bsc_ref[...]


def _ab_sc(x, wa, ba, wb, wsc, bsc):
    return pl.pallas_call(
        _ab_body,
        grid=(N // _RT,),
        in_specs=[pl.BlockSpec((_RT, D), lambda i: (i, 0))] +
                 [pl.BlockSpec((D, D), lambda i: (0, 0)),
                  pl.BlockSpec((1, D), lambda i: (0, 0)),
                  pl.BlockSpec((D, D), lambda i: (0, 0)),
                  pl.BlockSpec((D, D), lambda i: (0, 0)),
                  pl.BlockSpec((1, D), lambda i: (0, 0))],
        out_specs=tuple(pl.BlockSpec((_RT, D), lambda i: (i, 0))
                        for _ in range(3)),
        out_shape=tuple(jax.ShapeDtypeStruct((N, D), jnp.float32)
                        for _ in range(3)),
        interpret=_INTERPRET,
    )(x, wa, ba, wb, wsc, bsc)


_NWK = 32
_RPW = _NP // _NWK


_GCH = 128
_C0_SHARE_PCT = 75


def _sc_gather(idx, table):
    M = idx.shape[0]
    Wd = table.shape[1]
    rpw = M // _NWK
    nch = rpw // _GCH
    nb = 4 if (nch % 4 == 0 and Wd <= 128) else 2
    w0 = max(nb, ((2 * nch * _C0_SHARE_PCT // 100) // nb) * nb)

    def body(idx_hbm, table_hbm, out_hbm, idx_v, *rows_sem):
        rows, sem = rows_sem[:nb], rows_sem[nb]
        sid = lax.axis_index("s")
        cid = lax.axis_index("c")
        pair_first = sid * 2 * nch
        my_cnt = jnp.where(cid == 0, w0, 2 * nch - w0)
        my_loc = jnp.where(cid == 0, 0, w0)
        pltpu.sync_copy(idx_hbm.at[pl.ds(pair_first * _GCH, 2 * rpw)], idx_v)

        def macro(m):
            cps = []
            for b in range(nb):
                loc = (my_loc + m * nb + b) * _GCH
                cps.append(pltpu.async_copy(
                    table_hbm.at[idx_v.at[pl.ds(loc, _GCH)]],
                    rows[b], sem))
            for cp in cps:
                cp.wait()
            for b in range(nb):
                loc = (my_loc + m * nb + b) * _GCH
                pltpu.sync_copy(rows[b],
                                out_hbm.at[pl.ds(pair_first * _GCH + loc,
                                                 _GCH)])

        pl.loop(0, my_cnt // nb)(macro)

    mesh = plsc.VectorSubcoreMesh(core_axis_name="c", subcore_axis_name="s")
    f = pl.kernel(
        body,
        mesh=mesh,
        out_type=jax.ShapeDtypeStruct((M, Wd), jnp.float32),
        scratch_types=[pltpu.VMEM((2 * rpw,), jnp.int32)] +
                      [pltpu.VMEM((_GCH, Wd), jnp.float32)
                       for _ in range(nb)] +
                      [pltpu.SemaphoreType.DMA],
    )
    return f(idx, table)


def _econv_body(a_ref, bg_ref, sc_ref, w2_ref, b2_ref, w3_ref, b3_ref,
                s3_ref, t3_ref, out_ref):
    a = a_ref[...]
    acc = None
    for k in range(K):
        h = jnp.maximum(a + bg_ref[k], 0.0)
        h = jnp.maximum(jax.lax.dot_general(
            h, w2_ref[...], (((1,), (0,)), ((), ())),
            preferred_element_type=jnp.float32) + b2_ref[...], 0.0)
        h = jnp.maximum(jax.lax.dot_general(
            h, w3_ref[...], (((1,), (0,)), ((), ())),
            preferred_element_type=jnp.float32) + b3_ref[...], 0.0)
        h = h * s3_ref[...] + t3_ref[...]
        acc = h if acc is None else jnp.maximum(acc, h)
    out_ref[...] = acc + sc_ref[...]


def _econv(a, bg, sc, w2, b2, w3, b3, s3, t3):
    return pl.pallas_call(
        _econv_body,
        grid=(N // _RT,),
        in_specs=[
            pl.BlockSpec((_RT, D), lambda i: (i, 0)),
            pl.BlockSpec((K, _RT, D), lambda i: (0, i, 0)),
            pl.BlockSpec((_RT, D), lambda i: (i, 0)),
            pl.BlockSpec((D, D), lambda i: (0, 0)),
            pl.BlockSpec((1, D), lambda i: (0, 0)),
            pl.BlockSpec((D, D), lambda i: (0, 0)),
            pl.BlockSpec((1, D), lambda i: (0, 0)),
            pl.BlockSpec((1, D), lambda i: (0, 0)),
            pl.BlockSpec((1, D), lambda i: (0, 0)),
        ],
        out_specs=pl.BlockSpec((_RT, D), lambda i: (i, 0)),
        out_shape=jax.ShapeDtypeStruct((N, D), jnp.float32),
        interpret=_INTERPRET,
    )(a, bg, sc, w2, b2, w3, b3, s3, t3)


def _bn_fold(w, b):
    s = w / np.sqrt(1.0 + EPS)
    return s, b


def _dyn_edge_conv(x, batch, starts, p, c):
    nbr = _knn_pallas(x, batch, starts)
    s1, t1 = _bn_fold(p[c + 'bn1_w'], p[c + 'bn1_b'])
    s2, t2 = _bn_fold(p[c + 'bn2_w'], p[c + 'bn2_b'])
    s3, t3 = _bn_fold(p[c + 'bn3_w'], p[c + 'bn3_b'])
    ssc, tsc = _bn_fold(p[c + 'scbn_w'], p[c + 'scbn_b'])
    w1a = p[c + 'l1_w'][:D]
    w1b = p[c + 'l1_w'][D:]
    wa = w1a - w1b
    ba = p[c + 'l1_b'].reshape(1, H)
    w2f = s1[:, None] * p[c + 'l2_w']
    b2f = (t1 @ p[c + 'l2_w'] + p[c + 'l2_b']).reshape(1, H)
    w3f = s2[:, None] * p[c + 'l3_w']
    b3f = (t2 @ p[c + 'l3_w'] + p[c + 'l3_b']).reshape(1, H)
    wscf = p[c + 'sc_w'] * ssc[None, :]
    bscf = (p[c + 'sc_b'] * ssc + tsc).reshape(1, H)

    a, b, sc = _ab_sc(x, wa, ba, w1b, wscf, bscf)
    nbrT = jnp.zeros((K, _NP), jnp.int32).at[:, :N].set(nbr.T).reshape(-1)
    bg = _sc_gather(nbrT, b).reshape(K, _NP, D)
    return _econv(a, bg, sc, w2f, b2f, w3f, b3f,
                  s3.reshape(1, H), t3.reshape(1, H))


def _head_body(c1_ref, c2_ref, c3_ref, bcol_ref, gi_ref, w1g_ref, w1i_ref,
               b1_ref, w2_ref, b2_ref, wo_ref, bo_ref, out_ref):
    s = c1_ref[...] + c2_ref[...] + c3_ref[...]
    g_col = jax.lax.broadcasted_iota(jnp.int32, (G, 1), 0)
    ohT = (g_col == bcol_ref[...]).astype(jnp.float32)
    cnt = jnp.maximum(jnp.sum(ohT, axis=1, keepdims=True), 1.0)
    g = jax.lax.dot_general(ohT, s, (((1,), (0,)), ((), ())),
                            preferred_element_type=jnp.float32) / cnt
    g1 = jax.lax.dot_general(g, w1g_ref[...], (((1,), (0,)), ((), ())),
                             preferred_element_type=jnp.float32)
    g1 = g1 + jax.lax.dot_general(gi_ref[...], w1i_ref[...],
                                  (((1,), (0,)), ((), ())),
                                  preferred_element_type=jnp.float32)
    g1 = g1 + b1_ref[...]
    alpha = 1.6732632423543772
    scale = 1.0507009873554805
    g1 = scale * jnp.where(g1 > 0, g1,
                           alpha * (jnp.exp(jnp.minimum(g1, 0.0)) - 1.0))
    g2 = jax.lax.dot_general(g1, w2_ref[...], (((1,), (0,)), ((), ())),
                             preferred_element_type=jnp.float32) + b2_ref[...]
    g2 = scale * jnp.where(g2 > 0, g2,
                           alpha * (jnp.exp(jnp.minimum(g2, 0.0)) - 1.0))
    logits = jax.lax.dot_general(g2, wo_ref[...], (((1,), (0,)), ((), ())),
                                 preferred_element_type=jnp.float32) + bo_ref[...]
    lane = jax.lax.broadcasted_iota(jnp.int32, (G, 128), 1)
    ml = lane < C
    lm = jnp.where(ml, logits, -_INF)
    m = jnp.max(lm, axis=1, keepdims=True)
    ex = jnp.where(ml, jnp.exp(lm - m), 0.0)
    lse = jnp.log(jnp.sum(ex, axis=1, keepdims=True))
    out_ref[...] = logits - m - lse


def _head(c1, c2, c3, batch, graph_input, p):
    s0, t0 = _bn_fold(p['bn0_w'], p['bn0_b'])
    s0a, s0b = s0[:H], s0[H:]
    t0a, t0b = t0[:H], t0[H:]
    w1g = s0a[:, None] * p['d1_w'][:H]
    w1i = s0b[:, None] * p['d1_w'][H:]
    b1 = (t0a @ p['d1_w'][:H] + t0b @ p['d1_w'][H:] + p['d1_b']).reshape(1, H)
    w1i_p = jnp.zeros((128, H), jnp.float32).at[:GF].set(w1i)
    gi_p = jnp.zeros((G, 128), jnp.float32).at[:, :GF].set(graph_input)
    wo_p = jnp.zeros((H, 128), jnp.float32).at[:, :C].set(p['out_w'])
    bo_p = jnp.zeros((1, 128), jnp.float32).at[0, :C].set(p['out_b'])
    out = pl.pallas_call(
        _head_body,
        out_shape=jax.ShapeDtypeStruct((G, 128), jnp.float32),
        interpret=_INTERPRET,
    )(c1, c2, c3, batch.reshape(1, N), gi_p, w1g, w1i_p, b1,
      p['d2_w'], p['d2_b'].reshape(1, H), wo_p, bo_p)
    return out[:, :C]


_EP = 163840
_EPW = _EP // _NWK
_ECH = 128
_NCH = _EPW // _ECH
_ET = 2048
_AW = 144
_ISQH = float(1.0 / np.sqrt(float(H)))
E_REAL = 160000


def _edot_body(qd_ref, ks_ref, a_ref):
    a_ref[...] = jnp.sum(qd_ref[...] * ks_ref[...], axis=1, keepdims=True)


def _edot(qd, kvg):
    return pl.pallas_call(
        _edot_body,
        grid=(_EP // _ET,),
        in_specs=[pl.BlockSpec((_ET, H), lambda i: (i, 0)),
                  pl.BlockSpec((_ET, H), lambda i: (i, 0))],
        out_specs=pl.BlockSpec((_ET, 1), lambda i: (i, 0)),
        out_shape=jax.ShapeDtypeStruct((_EP, 1), jnp.float32),
        interpret=_INTERPRET,
    )(qd, kvg)


def _gmax_body(a_ref, out_ref):
    out_ref[...] = jnp.full((1, 128), jnp.max(a_ref[...]), jnp.float32)


def _gmax(a2d):
    return pl.pallas_call(
        _gmax_body,
        out_shape=jax.ShapeDtypeStruct((1, 128), jnp.float32),
        interpret=_INTERPRET,
    )(a2d)


def _st_body(a_ref, vs_ref, dm_ref, gm_ref, stn_ref, std_ref):
    i = pl.program_id(0)
    a = a_ref[...]
    ex = jnp.exp(a * _ISQH - gm_ref[0:1, 0:1])
    gid = i * _ET + jax.lax.broadcasted_iota(jnp.int32, (_ET, 1), 0)
    ex = jnp.where(gid < E_REAL, ex, 0.0)
    stn_ref[...] = ex * vs_ref[...]
    lane = jax.lax.broadcasted_iota(jnp.int32, (_ET, 128), 1)
    std_ref[...] = jnp.where(lane == dm_ref[...], ex, 0.0)


def _st_build(a, kvg, dmod, gm):
    return pl.pallas_call(
        _st_body,
        grid=(_EP // _ET,),
        in_specs=[pl.BlockSpec((_ET, 1), lambda i: (i, 0)),
                  pl.BlockSpec((_ET, H), lambda i: (i, 1)),
                  pl.BlockSpec((_ET, 1), lambda i: (i, 0)),
                  pl.BlockSpec((1, 128), lambda i: (0, 0))],
        out_specs=(pl.BlockSpec((_ET, H), lambda i: (i, 0)),
                   pl.BlockSpec((_ET, 128), lambda i: (i, 0))),
        out_shape=(jax.ShapeDtypeStruct((_EP, H), jnp.float32),
                   jax.ShapeDtypeStruct((_EP, 128), jnp.float32)),
        interpret=_INTERPRET,
    )(a, kvg, dmod, gm)


_DDIV = 128


def _sc_scatter_add(st, idx3, acc_rows):
    slab = acc_rows // 16

    def body(st_hbm, idx_hbm, z_hbm, acc_out, idxv, st0, st1, acc_sh, sem):
        cid = lax.axis_index("c")
        tid = lax.axis_index("s")
        wid = tid * 2 + cid
        base = wid * _EPW

        pltpu.sync_copy(z_hbm.at[pl.ds(0, slab)],
                        acc_sh.at[pl.ds(tid * slab, slab)])
        pltpu.sync_copy(idx_hbm.at[wid], idxv)
        plsc.subcore_barrier()

        sts = (st0, st1)

        def macro(m):
            cps = []
            for b in range(2):
                off = base + (m * 2 + b) * _ECH
                cps.append(pltpu.async_copy(st_hbm.at[pl.ds(off, _ECH)],
                                            sts[b], sem))
            cps[0].wait()
            pltpu.sync_copy(sts[0], acc_sh.at[idxv.at[m * 2]], add=True)
            cps[1].wait()
            pltpu.sync_copy(sts[1], acc_sh.at[idxv.at[m * 2 + 1]], add=True)

        pl.loop(0, _NCH // 2)(macro)
        plsc.subcore_barrier()
        pltpu.sync_copy(acc_sh.at[pl.ds(tid * slab, slab)],
                        acc_out.at[pl.ds(cid * acc_rows + tid * slab, slab)])

    z = jnp.zeros((max(_NP // 16, 128), 128), jnp.float32)
    mesh = plsc.VectorSubcoreMesh(core_axis_name="c", subcore_axis_name="s")
    f = pl.kernel(
        body,
        mesh=mesh,
        out_type=jax.ShapeDtypeStruct((2 * acc_rows, 128), jnp.float32),
        scratch_types=[
            pltpu.VMEM((_NCH, _ECH), jnp.int32),
            pltpu.VMEM((_ECH, 128), jnp.float32),
            pltpu.VMEM((_ECH, 128), jnp.float32),
            pltpu.VMEM_SHARED((acc_rows, 128), jnp.float32),
            pltpu.SemaphoreType.DMA,
        ],
    )
    return f(st, idx3, z)


def _attention_sc(q, kv, src, dst):
    srcp = jnp.zeros((_EP,), jnp.int32).at[:E_REAL].set(src)
    dstp = jnp.zeros((_EP,), jnp.int32).at[:E_REAL].set(dst)
    qd = _sc_gather(dstp, q)
    kvg = _sc_gather(srcp, kv)
    a = _edot(qd, kvg)
    gm = _gmax(a.reshape(_EP // 128, 128))
    stn, std = _st_build(a, kvg, (dstp % 128).reshape(_EP, 1), gm)
    accn = _sc_scatter_add(stn, dstp.reshape(_NWK, _NCH, _ECH), _NP)
    accd = _sc_scatter_add(std, (dstp // 128).reshape(_NWK, _NCH, _ECH),
                           _DDIV)
    num = accn.reshape(2, _NP, 128)
    den = (accd[:_DDIV] + accd[_DDIV:]).reshape(-1)[:_NP].reshape(_NP, 1)
    return num, den


def _gate_acc_body(acc_ref, den_ref, s_ref, wa_ref, wb_ref, c_ref):
    num = acc_ref[0] + acc_ref[1]
    den = den_ref[...]
    o = num / jnp.maximum(den, 1e-30)
    s = s_ref[...]
    z = jnp.sum(o * wa_ref[...] + s * wb_ref[...], axis=1, keepdims=True)
    bta = jax.nn.sigmoid(z)
    y = bta * s + (1.0 - bta) * o
    c_ref[...] = jnp.where(y > 0, y, jnp.exp(jnp.minimum(y, 0.0)) - 1.0)


def _gate_acc(num, den, skip, p):
    tb = p['tbeta_w']
    wa = (tb[0:128, 0] + tb[256:384, 0]).reshape(1, H)
    wb = (tb[128:256, 0] - tb[256:384, 0]).reshape(1, H)
    return pl.pallas_call(
        _gate_acc_body,
        grid=(N // _RT,),
        in_specs=[
            pl.BlockSpec((2, _RT, 128), lambda i: (0, i, 0)),
            pl.BlockSpec((_RT, 1), lambda i: (i, 0)),
            pl.BlockSpec((_RT, H), lambda i: (i, 0)),
            pl.BlockSpec((1, H), lambda i: (0, 0)),
            pl.BlockSpec((1, H), lambda i: (0, 0)),
        ],
        out_specs=pl.BlockSpec((_RT, H), lambda i: (i, 0)),
        out_shape=jax.ShapeDtypeStruct((N, H), jnp.float32),
        interpret=_INTERPRET,
    )(num, den, skip, wa, wb)


def kernel(x, edge_index, graph_input, batch, params):
    src, dst = edge_index[0], edge_index[1]
    h, starts = _graph_norm_starts(x, batch, params)
    q, kv, skip = _projections(h, params)
    num, den = _attention_sc(q, kv, src, dst)
    c1 = _gate_acc(num, den, skip, params)
    c2 = _dyn_edge_conv(c1, batch, starts, params, 'c2_')
    c3 = _dyn_edge_conv(c2, batch, starts, params, 'c3_')
    return _head(c1, c2, c3, batch, graph_input, params)
